# baseline (speedup 1.0000x reference)
import jax
import jax.numpy as jnp
from jax.experimental import pallas as pl

_U = 50000
_I = 50000
_D = 32
_B = 16384
_T = 1.0


def _finalize_body(siu, sju, du, sii, sji, di, out):
    def side(si, sj, dg, n):
        num = jnp.exp(si / _T)
        den = jnp.exp(sj / _T) + num
        return -jnp.sum(jnp.log(num / den) * dg) / n

    total = side(siu[...], sju[...], du[...], _U) + side(
        sii[...], sji[...], di[...], _I)
    out[...] = jnp.reshape(total, (1, 1))


def kernel(user, item_i, item_j, degree_U, item_z_U, user_, item_i_, item_j_,
           degree_I, item_z_I, embed_user, embed_item, old_U_emb, old_I_emb,
           ui_rows, ui_cols, ui_vals):
    def spmm_ui(x):
        return jax.ops.segment_sum(
            ui_vals[:, None] * jnp.take(x, ui_cols, axis=0), ui_rows,
            num_segments=_U)

    def spmm_iu(x):
        return jax.ops.segment_sum(
            ui_vals[:, None] * jnp.take(x, ui_rows, axis=0), ui_cols,
            num_segments=_I)

    g1u = spmm_ui(embed_item)
    g1i = spmm_iu(embed_user)
    g2u = spmm_ui(g1i)
    g2i = spmm_iu(g1u)
    g3u = spmm_ui(g2i)
    g3i = spmm_iu(g2u)
    gcn_users = embed_user + 0.5 * g1u + (1.0 / 3.0) * g2u + 0.25 * g3u
    gcn_items = embed_item + 0.5 * g1i + (1.0 / 3.0) * g2i + 0.25 * g3i

    def dots(idx_u, idx_i, idx_j, emb_u, emb_i):
        u = jnp.take(emb_u, idx_u, axis=0)
        i = jnp.take(emb_i, idx_i, axis=0)
        j = jnp.take(emb_i, idx_j, axis=0)
        return jnp.sum(u * i, axis=-1), jnp.sum(u * j, axis=-1)

    siu, sju = dots(user, item_i, item_j, old_U_emb, gcn_users)
    sii, sji = dots(user_, item_i_, item_j_, old_I_emb, gcn_items)

    r = lambda a: a.reshape(128, 128)
    out = pl.pallas_call(
        _finalize_body,
        out_shape=jax.ShapeDtypeStruct((1, 1), jnp.float32),
    )(r(siu), r(sju), r(degree_U), r(sii), r(sji), r(degree_I))
    return out.reshape(1)


# speedup vs baseline: 8.4670x; 8.4670x over previous
import functools

import jax
import jax.numpy as jnp
from jax import lax
from jax.experimental import pallas as pl
from jax.experimental.pallas import tpu as pltpu
from jax.experimental.pallas import tpu_sc as plsc

_N = 50000
_D = 32
_E = 1600000
_B = 16384
_T = 1.0

_SUB = 128
_NSUB = 4
_W = _SUB * _NSUB
_WINS = 200
_EPAD = 16 * _WINS * _W
_EROWS = _EPAD // _SUB

_RCH = 128
_NCHUNK = _N // _RCH
_NTAIL = _N - _NCHUNK * _RCH

_CH = 128
_CPT = _B // 16 // _CH

_mesh = plsc.VectorSubcoreMesh(core_axis_name="c", subcore_axis_name="s")


def _spmm_body(tabs, gidx, sidx, vals, out, acc, cbuf, sbuf, vbuf, gbuf, sem):
    c = lax.axis_index("c")
    s = lax.axis_index("s")

    def zrow(k, carry):
        gbuf[0, k, pl.ds(0, 16)] = jnp.zeros((16,), jnp.float32)
        gbuf[0, k, pl.ds(16, 16)] = jnp.zeros((16,), jnp.float32)
        return carry

    lax.fori_loop(0, _SUB, zrow, 0)

    def zcp(i, carry):
        cid = s + 16 * i

        @pl.when(cid < _NCHUNK)
        def _():
            r = pl.multiple_of(cid * _RCH, 8)
            pltpu.sync_copy(gbuf.at[0], acc.at[pl.ds(r, _RCH)])

        @pl.when(cid == _NCHUNK)
        def _():
            pltpu.sync_copy(gbuf.at[0, pl.ds(0, _NTAIL)],
                            acc.at[pl.ds(_NCHUNK * _RCH, _NTAIL)])

        return carry

    lax.fori_loop(0, 25, zcp, 0)
    plsc.subcore_barrier()

    def wbody(w, carry):
        r0 = pl.multiple_of((c * 16 + s) * _WINS * _NSUB + w * _NSUB, 4)
        pltpu.sync_copy(gidx.at[pl.ds(r0, _NSUB)], cbuf)
        pltpu.sync_copy(sidx.at[pl.ds(r0, _NSUB)], sbuf)
        rv = pl.multiple_of(s * _WINS * _NSUB + w * _NSUB, 4)
        pltpu.sync_copy(vals.at[pl.ds(rv, _NSUB)], vbuf)
        cps = [pltpu.async_copy(tabs.at[cbuf.at[j]], gbuf.at[j], sem)
               for j in range(_NSUB)]
        for cp in cps:
            cp.wait()
        for j in range(_NSUB):
            def scale(k, carry2, j=j):
                kv = jnp.full((16,), 0, jnp.int32) + k
                v = plsc.load_gather(vbuf.at[j], [kv])
                gbuf[j, k, pl.ds(0, 16)] = gbuf[j, k, pl.ds(0, 16)] * v
                gbuf[j, k, pl.ds(16, 16)] = gbuf[j, k, pl.ds(16, 16)] * v
                return carry2

            lax.fori_loop(0, _SUB, scale, 0)
        for j in range(_NSUB):
            pltpu.sync_copy(gbuf.at[j], acc.at[sbuf.at[j]], add=True)
        return carry

    lax.fori_loop(0, _WINS, wbody, 0)
    plsc.subcore_barrier()

    def obody(i, carry):
        cid = s + 16 * i

        @pl.when(cid < _NCHUNK)
        def _():
            r = pl.multiple_of(cid * _RCH, 8)
            ro = pl.multiple_of(c * _N + cid * _RCH, 8)
            pltpu.sync_copy(acc.at[pl.ds(r, _RCH)], gbuf.at[0])
            pltpu.sync_copy(gbuf.at[0], out.at[pl.ds(ro, _RCH)])

        @pl.when(cid == _NCHUNK)
        def _():
            ro = pl.multiple_of(c * _N + _NCHUNK * _RCH, 8)
            pltpu.sync_copy(acc.at[pl.ds(_NCHUNK * _RCH, _NTAIL)],
                            gbuf.at[0, pl.ds(0, _NTAIL)])
            pltpu.sync_copy(gbuf.at[0, pl.ds(0, _NTAIL)],
                            out.at[pl.ds(ro, _NTAIL)])

        return carry

    lax.fori_loop(0, 25, obody, 0)


_spmm = functools.partial(
    pl.kernel,
    out_type=jax.ShapeDtypeStruct((2 * _N, _D), jnp.float32),
    mesh=_mesh,
    scratch_types=[
        pltpu.VMEM_SHARED((_N, _D), jnp.float32),
        pltpu.VMEM((_NSUB, _SUB), jnp.int32),
        pltpu.VMEM((_NSUB, _SUB), jnp.int32),
        pltpu.VMEM((_NSUB, _SUB), jnp.float32),
        pltpu.VMEM((_NSUB, _SUB, _D), jnp.float32),
        pltpu.SemaphoreType.DMA,
    ],
    compiler_params=pltpu.CompilerParams(use_tc_tiling_on_sc=False,
                                         needs_layout_passes=False),
)(_spmm_body)


def _gather_body(olds, e0, g1, g2, g3, uidx, iidx, jidx, out,
                 ubi, ibi, jbi, ub, i0, i1, i2, i3, j0, j1, j2, j3, sem):
    c = lax.axis_index("c")
    s = lax.axis_index("s")
    r0 = pl.multiple_of(c * (_B // _CH) + s * _CPT, 8)
    pltpu.sync_copy(uidx.at[pl.ds(r0, _CPT)], ubi)
    pltpu.sync_copy(iidx.at[pl.ds(r0, _CPT)], ibi)
    pltpu.sync_copy(jidx.at[pl.ds(r0, _CPT)], jbi)

    def chunk(jj, carry):
        cps = [
            pltpu.async_copy(olds.at[ubi.at[jj]], ub, sem),
            pltpu.async_copy(e0.at[ibi.at[jj]], i0, sem),
            pltpu.async_copy(g1.at[ibi.at[jj]], i1, sem),
            pltpu.async_copy(g2.at[ibi.at[jj]], i2, sem),
            pltpu.async_copy(g3.at[ibi.at[jj]], i3, sem),
            pltpu.async_copy(e0.at[jbi.at[jj]], j0, sem),
            pltpu.async_copy(g1.at[jbi.at[jj]], j1, sem),
            pltpu.async_copy(g2.at[jbi.at[jj]], j2, sem),
            pltpu.async_copy(g3.at[jbi.at[jj]], j3, sem),
        ]
        for cp in cps:
            cp.wait()

        def comb(k, carry2):
            for sl in (pl.ds(0, 16), pl.ds(16, 16)):
                i0[k, sl] = (i0[k, sl] + 0.5 * i1[k, sl]
                             + (1.0 / 3.0) * i2[k, sl] + 0.25 * i3[k, sl])
                j0[k, sl] = (j0[k, sl] + 0.5 * j1[k, sl]
                             + (1.0 / 3.0) * j2[k, sl] + 0.25 * j3[k, sl])
            return carry2

        lax.fori_loop(0, _CH, comb, 0)
        b0 = (s * _CPT + jj) * _CH
        ou = pl.multiple_of((c * 3 + 0) * _B + b0, 8)
        oi = pl.multiple_of((c * 3 + 1) * _B + b0, 8)
        oj = pl.multiple_of((c * 3 + 2) * _B + b0, 8)
        pltpu.sync_copy(ub, out.at[pl.ds(ou, _CH)])
        pltpu.sync_copy(i0, out.at[pl.ds(oi, _CH)])
        pltpu.sync_copy(j0, out.at[pl.ds(oj, _CH)])
        return carry

    lax.fori_loop(0, _CPT, chunk, 0)


_gather = functools.partial(
    pl.kernel,
    out_type=jax.ShapeDtypeStruct((6 * _B, _D), jnp.float32),
    mesh=_mesh,
    scratch_types=(
        [pltpu.VMEM((_CPT, _CH), jnp.int32)] * 3
        + [pltpu.VMEM((_CH, _D), jnp.float32)] * 9
        + [pltpu.SemaphoreType.DMA]
    ),
    compiler_params=pltpu.CompilerParams(use_tc_tiling_on_sc=False,
                                         needs_layout_passes=False),
)(_gather_body)


_FB = 2048


def _fin_body(g_ref, du_ref, di_ref, out_ref):
    pid = pl.program_id(0)
    g = g_ref[...]
    u = g[:, 0]
    i = g[:, 1]
    j = g[:, 2]
    si = jnp.sum(u * i, axis=-1) / _T
    sj = jnp.sum(u * j, axis=-1) / _T
    num = jnp.exp(si)
    den = jnp.exp(sj) + num
    deg = jnp.stack([du_ref[...].reshape(_FB), di_ref[...].reshape(_FB)])
    part = -jnp.sum(jnp.log(num / den) * deg) / _N

    @pl.when(pid == 0)
    def _():
        out_ref[...] = jnp.zeros((1, 1), jnp.float32)

    out_ref[...] += jnp.reshape(part, (1, 1))


def kernel(user, item_i, item_j, degree_U, item_z_U, user_, item_i_, item_j_,
           degree_I, item_z_I, embed_user, embed_item, old_U_emb, old_I_emb,
           ui_rows, ui_cols, ui_vals):
    rows = ui_rows.astype(jnp.int32)
    cols = ui_cols.astype(jnp.int32)
    npad = _EPAD - _E
    pad_idx = jnp.arange(npad, dtype=jnp.int32) % _N
    pad_val = jnp.zeros((npad,), jnp.float32)
    cat = jnp.concatenate
    g_side0 = cat([cols + _N, pad_idx + _N])
    g_side1 = cat([rows, pad_idx])
    s_side0 = cat([rows, pad_idx])
    s_side1 = cat([cols, pad_idx])
    gidx = cat([g_side0, g_side1]).reshape(2 * _EROWS, _SUB)
    sidx = cat([s_side0, s_side1]).reshape(2 * _EROWS, _SUB)
    vals = cat([ui_vals.astype(jnp.float32), pad_val]).reshape(_EROWS, _SUB)

    e0 = cat([embed_user, embed_item])
    g1 = _spmm(e0, gidx, sidx, vals)
    g2 = _spmm(g1, gidx, sidx, vals)
    g3 = _spmm(g2, gidx, sidx, vals)

    olds = cat([old_U_emb, old_I_emb])
    i32 = jnp.int32

    def bidx(a, b):
        return cat([a.astype(i32), b.astype(i32) + _N]).reshape(
            2 * (_B // _CH), _CH)

    uidx = bidx(user, user_)
    iidx = bidx(item_i, item_i_)
    jidx = bidx(item_j, item_j_)
    gath = _gather(olds, e0, g1, g2, g3, uidx, iidx, jidx)

    nfb = _B // _FB
    dshape = (nfb, 16, _FB // 16)
    out = pl.pallas_call(
        _fin_body,
        grid=(nfb,),
        in_specs=[
            pl.BlockSpec((2, 3, _FB, _D), lambda b: (0, 0, b, 0)),
            pl.BlockSpec((1, 16, _FB // 16), lambda b: (b, 0, 0)),
            pl.BlockSpec((1, 16, _FB // 16), lambda b: (b, 0, 0)),
        ],
        out_specs=pl.BlockSpec((1, 1), lambda b: (0, 0)),
        out_shape=jax.ShapeDtypeStruct((1, 1), jnp.float32),
    )(gath.reshape(2, 3, _B, _D), degree_U.reshape(dshape),
      degree_I.reshape(dshape))
    return out.reshape(1)


# speedup vs baseline: 9.1146x; 1.0765x over previous
import functools

import jax
import jax.numpy as jnp
from jax import lax
from jax.experimental import pallas as pl
from jax.experimental.pallas import tpu as pltpu
from jax.experimental.pallas import tpu_sc as plsc

_N = 50000
_D = 32
_E = 1600000
_B = 16384
_T = 1.0

_SUB = 128
_NSUB = 4
_W = _SUB * _NSUB
_WINS = 200
_EPAD = 16 * _WINS * _W
_EROWS = _EPAD // _SUB

_RCH = 128
_NCHUNK = _N // _RCH
_NTAIL = _N - _NCHUNK * _RCH

_CH = 128
_CPT = _B // 16 // _CH

_mesh = plsc.VectorSubcoreMesh(core_axis_name="c", subcore_axis_name="s")


def _spmm_body(tabs, gidx, sidx, vals, out, acc, cbuf, sbuf, vbuf, gbuf, sem):
    c = lax.axis_index("c")
    s = lax.axis_index("s")

    def zrow(k, carry):
        gbuf[0, k, pl.ds(0, 16)] = jnp.zeros((16,), jnp.float32)
        gbuf[0, k, pl.ds(16, 16)] = jnp.zeros((16,), jnp.float32)
        return carry

    lax.fori_loop(0, _SUB, zrow, 0)

    def zcp(i, carry):
        cid = s + 16 * i

        @pl.when(cid < _NCHUNK)
        def _():
            r = pl.multiple_of(cid * _RCH, 8)
            pltpu.sync_copy(gbuf.at[0], acc.at[pl.ds(r, _RCH)])

        @pl.when(cid == _NCHUNK)
        def _():
            pltpu.sync_copy(gbuf.at[0, pl.ds(0, _NTAIL)],
                            acc.at[pl.ds(_NCHUNK * _RCH, _NTAIL)])

        return carry

    lax.fori_loop(0, 25, zcp, 0)
    plsc.subcore_barrier()

    def wbody(w, carry):
        r0 = pl.multiple_of((c * 16 + s) * _WINS * _NSUB + w * _NSUB, 4)
        pltpu.sync_copy(gidx.at[pl.ds(r0, _NSUB)], cbuf)
        pltpu.sync_copy(sidx.at[pl.ds(r0, _NSUB)], sbuf)
        rv = pl.multiple_of(s * _WINS * _NSUB + w * _NSUB, 4)
        pltpu.sync_copy(vals.at[pl.ds(rv, _NSUB)], vbuf)
        cps = [pltpu.async_copy(tabs.at[cbuf.at[j]], gbuf.at[j], sem)
               for j in range(_NSUB)]
        for cp in cps:
            cp.wait()
        for j in range(_NSUB):
            def scale(k, carry2, j=j):
                kv = jnp.full((16,), 0, jnp.int32) + k
                v = plsc.load_gather(vbuf.at[j], [kv])
                gbuf[j, k, pl.ds(0, 16)] = gbuf[j, k, pl.ds(0, 16)] * v
                gbuf[j, k, pl.ds(16, 16)] = gbuf[j, k, pl.ds(16, 16)] * v
                return carry2

            lax.fori_loop(0, _SUB, scale, 0, unroll=8)
        for j in range(_NSUB):
            pltpu.sync_copy(gbuf.at[j], acc.at[sbuf.at[j]], add=True)
        return carry

    lax.fori_loop(0, _WINS, wbody, 0)
    plsc.subcore_barrier()

    def obody(i, carry):
        cid = s + 16 * i

        @pl.when(cid < _NCHUNK)
        def _():
            r = pl.multiple_of(cid * _RCH, 8)
            ro = pl.multiple_of(c * _N + cid * _RCH, 8)
            pltpu.sync_copy(acc.at[pl.ds(r, _RCH)], gbuf.at[0])
            pltpu.sync_copy(gbuf.at[0], out.at[pl.ds(ro, _RCH)])

        @pl.when(cid == _NCHUNK)
        def _():
            ro = pl.multiple_of(c * _N + _NCHUNK * _RCH, 8)
            pltpu.sync_copy(acc.at[pl.ds(_NCHUNK * _RCH, _NTAIL)],
                            gbuf.at[0, pl.ds(0, _NTAIL)])
            pltpu.sync_copy(gbuf.at[0, pl.ds(0, _NTAIL)],
                            out.at[pl.ds(ro, _NTAIL)])

        return carry

    lax.fori_loop(0, 25, obody, 0)


_spmm = functools.partial(
    pl.kernel,
    out_type=jax.ShapeDtypeStruct((2 * _N, _D), jnp.float32),
    mesh=_mesh,
    scratch_types=[
        pltpu.VMEM_SHARED((_N, _D), jnp.float32),
        pltpu.VMEM((_NSUB, _SUB), jnp.int32),
        pltpu.VMEM((_NSUB, _SUB), jnp.int32),
        pltpu.VMEM((_NSUB, _SUB), jnp.float32),
        pltpu.VMEM((_NSUB, _SUB, _D), jnp.float32),
        pltpu.SemaphoreType.DMA,
    ],
    compiler_params=pltpu.CompilerParams(use_tc_tiling_on_sc=False,
                                         needs_layout_passes=False),
)(_spmm_body)


def _gather_body(olds, e0, g1, g2, g3, uidx, iidx, jidx, out,
                 ubi, ibi, jbi, ub, i0, i1, i2, i3, j0, j1, j2, j3, sem):
    c = lax.axis_index("c")
    s = lax.axis_index("s")
    r0 = pl.multiple_of(c * (_B // _CH) + s * _CPT, 8)
    pltpu.sync_copy(uidx.at[pl.ds(r0, _CPT)], ubi)
    pltpu.sync_copy(iidx.at[pl.ds(r0, _CPT)], ibi)
    pltpu.sync_copy(jidx.at[pl.ds(r0, _CPT)], jbi)

    def chunk(jj, carry):
        cps = [
            pltpu.async_copy(olds.at[ubi.at[jj]], ub, sem),
            pltpu.async_copy(e0.at[ibi.at[jj]], i0, sem),
            pltpu.async_copy(g1.at[ibi.at[jj]], i1, sem),
            pltpu.async_copy(g2.at[ibi.at[jj]], i2, sem),
            pltpu.async_copy(g3.at[ibi.at[jj]], i3, sem),
            pltpu.async_copy(e0.at[jbi.at[jj]], j0, sem),
            pltpu.async_copy(g1.at[jbi.at[jj]], j1, sem),
            pltpu.async_copy(g2.at[jbi.at[jj]], j2, sem),
            pltpu.async_copy(g3.at[jbi.at[jj]], j3, sem),
        ]
        for cp in cps:
            cp.wait()

        def comb(k, carry2):
            for sl in (pl.ds(0, 16), pl.ds(16, 16)):
                i0[k, sl] = (i0[k, sl] + 0.5 * i1[k, sl]
                             + (1.0 / 3.0) * i2[k, sl] + 0.25 * i3[k, sl])
                j0[k, sl] = (j0[k, sl] + 0.5 * j1[k, sl]
                             + (1.0 / 3.0) * j2[k, sl] + 0.25 * j3[k, sl])
            return carry2

        lax.fori_loop(0, _CH, comb, 0)
        b0 = (s * _CPT + jj) * _CH
        ou = pl.multiple_of((c * 3 + 0) * _B + b0, 8)
        oi = pl.multiple_of((c * 3 + 1) * _B + b0, 8)
        oj = pl.multiple_of((c * 3 + 2) * _B + b0, 8)
        pltpu.sync_copy(ub, out.at[pl.ds(ou, _CH)])
        pltpu.sync_copy(i0, out.at[pl.ds(oi, _CH)])
        pltpu.sync_copy(j0, out.at[pl.ds(oj, _CH)])
        return carry

    lax.fori_loop(0, _CPT, chunk, 0)


_gather = functools.partial(
    pl.kernel,
    out_type=jax.ShapeDtypeStruct((6 * _B, _D), jnp.float32),
    mesh=_mesh,
    scratch_types=(
        [pltpu.VMEM((_CPT, _CH), jnp.int32)] * 3
        + [pltpu.VMEM((_CH, _D), jnp.float32)] * 9
        + [pltpu.SemaphoreType.DMA]
    ),
    compiler_params=pltpu.CompilerParams(use_tc_tiling_on_sc=False,
                                         needs_layout_passes=False),
)(_gather_body)


_FB = 2048


def _fin_body(g_ref, du_ref, di_ref, out_ref):
    pid = pl.program_id(0)
    g = g_ref[...]
    u = g[:, 0]
    i = g[:, 1]
    j = g[:, 2]
    si = jnp.sum(u * i, axis=-1) / _T
    sj = jnp.sum(u * j, axis=-1) / _T
    num = jnp.exp(si)
    den = jnp.exp(sj) + num
    deg = jnp.stack([du_ref[...].reshape(_FB), di_ref[...].reshape(_FB)])
    part = -jnp.sum(jnp.log(num / den) * deg) / _N

    @pl.when(pid == 0)
    def _():
        out_ref[...] = jnp.zeros((1, 1), jnp.float32)

    out_ref[...] += jnp.reshape(part, (1, 1))


def kernel(user, item_i, item_j, degree_U, item_z_U, user_, item_i_, item_j_,
           degree_I, item_z_I, embed_user, embed_item, old_U_emb, old_I_emb,
           ui_rows, ui_cols, ui_vals):
    rows = ui_rows.astype(jnp.int32)
    cols = ui_cols.astype(jnp.int32)
    npad = _EPAD - _E
    pad_idx = jnp.arange(npad, dtype=jnp.int32) % _N
    pad_val = jnp.zeros((npad,), jnp.float32)
    cat = jnp.concatenate
    g_side0 = cat([cols + _N, pad_idx + _N])
    g_side1 = cat([rows, pad_idx])
    s_side0 = cat([rows, pad_idx])
    s_side1 = cat([cols, pad_idx])
    gidx = cat([g_side0, g_side1]).reshape(2 * _EROWS, _SUB)
    sidx = cat([s_side0, s_side1]).reshape(2 * _EROWS, _SUB)
    vals = cat([ui_vals.astype(jnp.float32), pad_val]).reshape(_EROWS, _SUB)

    e0 = cat([embed_user, embed_item])
    g1 = _spmm(e0, gidx, sidx, vals)
    g2 = _spmm(g1, gidx, sidx, vals)
    g3 = _spmm(g2, gidx, sidx, vals)

    olds = cat([old_U_emb, old_I_emb])
    i32 = jnp.int32

    def bidx(a, b):
        return cat([a.astype(i32), b.astype(i32) + _N]).reshape(
            2 * (_B // _CH), _CH)

    uidx = bidx(user, user_)
    iidx = bidx(item_i, item_i_)
    jidx = bidx(item_j, item_j_)
    gath = _gather(olds, e0, g1, g2, g3, uidx, iidx, jidx)

    nfb = _B // _FB
    dshape = (nfb, 16, _FB // 16)
    out = pl.pallas_call(
        _fin_body,
        grid=(nfb,),
        in_specs=[
            pl.BlockSpec((2, 3, _FB, _D), lambda b: (0, 0, b, 0)),
            pl.BlockSpec((1, 16, _FB // 16), lambda b: (b, 0, 0)),
            pl.BlockSpec((1, 16, _FB // 16), lambda b: (b, 0, 0)),
        ],
        out_specs=pl.BlockSpec((1, 1), lambda b: (0, 0)),
        out_shape=jax.ShapeDtypeStruct((1, 1), jnp.float32),
    )(gath.reshape(2, 3, _B, _D), degree_U.reshape(dshape),
      degree_I.reshape(dshape))
    return out.reshape(1)


# speedup vs baseline: 14.1556x; 1.5531x over previous
import functools

import jax
import jax.numpy as jnp
from jax import lax
from jax.experimental import pallas as pl
from jax.experimental.pallas import tpu as pltpu
from jax.experimental.pallas import tpu_sc as plsc

_N = 50000
_D = 32
_E = 1600000
_B = 16384
_T = 1.0

_SUB = 128
_NSUB = 2
_W = _SUB * _NSUB
_WINS = 400
_EPAD = 16 * _WINS * _W
_EROWS = _EPAD // _SUB

_RCH = 128
_NCHUNK = _N // _RCH
_NTAIL = _N - _NCHUNK * _RCH

_CH = 128
_CPT = _B // 16 // _CH

_mesh = plsc.VectorSubcoreMesh(core_axis_name="c", subcore_axis_name="s")


def _spmm_body(tabs, gidx, sidx, vals, out, acc, cbuf, sbuf, vbuf, gbuf, sem):
    c = lax.axis_index("c")
    s = lax.axis_index("s")

    def zrow(k, carry):
        gbuf[0, 0, k, pl.ds(0, 16)] = jnp.zeros((16,), jnp.float32)
        gbuf[0, 0, k, pl.ds(16, 16)] = jnp.zeros((16,), jnp.float32)
        return carry

    lax.fori_loop(0, _SUB, zrow, 0)

    def zcp(i, carry):
        cid = s + 16 * i

        @pl.when(cid < _NCHUNK)
        def _():
            r = pl.multiple_of(cid * _RCH, 8)
            pltpu.sync_copy(gbuf.at[0, 0], acc.at[pl.ds(r, _RCH)])

        @pl.when(cid == _NCHUNK)
        def _():
            pltpu.sync_copy(gbuf.at[0, 0, pl.ds(0, _NTAIL)],
                            acc.at[pl.ds(_NCHUNK * _RCH, _NTAIL)])

        return carry

    lax.fori_loop(0, 25, zcp, 0)
    plsc.subcore_barrier()

    sem_g = (sem.at[0], sem.at[1])
    sem_s = (sem.at[2], sem.at[3])
    sem_i = (sem.at[4], sem.at[5])

    def idx_fire(w, sl):
        r0 = pl.multiple_of((c * 16 + s) * _WINS * _NSUB + w * _NSUB, 2)
        rv = pl.multiple_of(s * _WINS * _NSUB + w * _NSUB, 2)
        pltpu.async_copy(gidx.at[pl.ds(r0, _NSUB)], cbuf.at[sl], sem_i[sl])
        pltpu.async_copy(sidx.at[pl.ds(r0, _NSUB)], sbuf.at[sl], sem_i[sl])
        pltpu.async_copy(vals.at[pl.ds(rv, _NSUB)], vbuf.at[sl], sem_i[sl])

    def idx_drain(sl):
        pltpu.make_async_copy(gidx.at[pl.ds(0, _NSUB)], cbuf.at[sl],
                              sem_i[sl]).wait()
        pltpu.make_async_copy(sidx.at[pl.ds(0, _NSUB)], sbuf.at[sl],
                              sem_i[sl]).wait()
        pltpu.make_async_copy(vals.at[pl.ds(0, _NSUB)], vbuf.at[sl],
                              sem_i[sl]).wait()

    def gat_fire(sl):
        for j in range(_NSUB):
            pltpu.async_copy(tabs.at[cbuf.at[sl, j]], gbuf.at[sl, j],
                             sem_g[sl])

    def gat_drain(sl):
        for j in range(_NSUB):
            pltpu.make_async_copy(tabs.at[cbuf.at[sl, j]], gbuf.at[sl, j],
                                  sem_g[sl]).wait()

    def sca_fire(sl):
        for j in range(_NSUB):
            pltpu.async_copy(gbuf.at[sl, j], acc.at[sbuf.at[sl, j]],
                             sem_s[sl], add=True)

    def sca_drain(sl):
        for j in range(_NSUB):
            pltpu.make_async_copy(gbuf.at[sl, j], acc.at[sbuf.at[sl, j]],
                                  sem_s[sl]).wait()

    def scale(sl):
        for j in range(_NSUB):
            def sbody(k, carry2, j=j):
                kv = jnp.full((16,), 0, jnp.int32) + k
                v = plsc.load_gather(vbuf.at[sl, j], [kv])
                gbuf[sl, j, k, pl.ds(0, 16)] = gbuf[sl, j, k, pl.ds(0, 16)] * v
                gbuf[sl, j, k, pl.ds(16, 16)] = (
                    gbuf[sl, j, k, pl.ds(16, 16)] * v)
                return carry2

            lax.fori_loop(0, _SUB, sbody, 0, unroll=8)

    def process(w, cur, nxt):
        @pl.when(w + 1 < _WINS)
        def _():
            idx_fire(w + 1, nxt)

        @pl.when(w >= 1)
        def _():
            sca_drain(nxt)

        @pl.when(w + 1 < _WINS)
        def _():
            idx_drain(nxt)
            gat_fire(nxt)

        gat_drain(cur)
        scale(cur)
        sca_fire(cur)

    idx_fire(0, 0)
    idx_drain(0)
    gat_fire(0)

    def wpair(i, carry):
        process(2 * i, 0, 1)
        process(2 * i + 1, 1, 0)
        return carry

    lax.fori_loop(0, _WINS // 2, wpair, 0)
    sca_drain(1)
    plsc.subcore_barrier()

    def obody(i, carry):
        cid = s + 16 * i

        @pl.when(cid < _NCHUNK)
        def _():
            r = pl.multiple_of(cid * _RCH, 8)
            ro = pl.multiple_of(c * _N + cid * _RCH, 8)
            pltpu.sync_copy(acc.at[pl.ds(r, _RCH)], gbuf.at[0, 0])
            pltpu.sync_copy(gbuf.at[0, 0], out.at[pl.ds(ro, _RCH)])

        @pl.when(cid == _NCHUNK)
        def _():
            ro = pl.multiple_of(c * _N + _NCHUNK * _RCH, 8)
            pltpu.sync_copy(acc.at[pl.ds(_NCHUNK * _RCH, _NTAIL)],
                            gbuf.at[0, 0, pl.ds(0, _NTAIL)])
            pltpu.sync_copy(gbuf.at[0, 0, pl.ds(0, _NTAIL)],
                            out.at[pl.ds(ro, _NTAIL)])

        return carry

    lax.fori_loop(0, 25, obody, 0)


_spmm = functools.partial(
    pl.kernel,
    out_type=jax.ShapeDtypeStruct((2 * _N, _D), jnp.float32),
    mesh=_mesh,
    scratch_types=[
        pltpu.VMEM_SHARED((_N, _D), jnp.float32),
        pltpu.VMEM((2, _NSUB, _SUB), jnp.int32),
        pltpu.VMEM((2, _NSUB, _SUB), jnp.int32),
        pltpu.VMEM((2, _NSUB, _SUB), jnp.float32),
        pltpu.VMEM((2, _NSUB, _SUB, _D), jnp.float32),
        pltpu.SemaphoreType.DMA((6,)),
    ],
    compiler_params=pltpu.CompilerParams(use_tc_tiling_on_sc=False,
                                         needs_layout_passes=False),
)(_spmm_body)


def _gather_body(olds, e0, g1, g2, g3, uidx, iidx, jidx, out,
                 ubi, ibi, jbi, ub, i0, i1, i2, i3, j0, j1, j2, j3, sem):
    c = lax.axis_index("c")
    s = lax.axis_index("s")
    r0 = pl.multiple_of(c * (_B // _CH) + s * _CPT, 8)
    pltpu.sync_copy(uidx.at[pl.ds(r0, _CPT)], ubi)
    pltpu.sync_copy(iidx.at[pl.ds(r0, _CPT)], ibi)
    pltpu.sync_copy(jidx.at[pl.ds(r0, _CPT)], jbi)

    def chunk(jj, carry):
        cps = [
            pltpu.async_copy(olds.at[ubi.at[jj]], ub, sem),
            pltpu.async_copy(e0.at[ibi.at[jj]], i0, sem),
            pltpu.async_copy(g1.at[ibi.at[jj]], i1, sem),
            pltpu.async_copy(g2.at[ibi.at[jj]], i2, sem),
            pltpu.async_copy(g3.at[ibi.at[jj]], i3, sem),
            pltpu.async_copy(e0.at[jbi.at[jj]], j0, sem),
            pltpu.async_copy(g1.at[jbi.at[jj]], j1, sem),
            pltpu.async_copy(g2.at[jbi.at[jj]], j2, sem),
            pltpu.async_copy(g3.at[jbi.at[jj]], j3, sem),
        ]
        for cp in cps:
            cp.wait()

        def comb(k, carry2):
            for sl in (pl.ds(0, 16), pl.ds(16, 16)):
                i0[k, sl] = (i0[k, sl] + 0.5 * i1[k, sl]
                             + (1.0 / 3.0) * i2[k, sl] + 0.25 * i3[k, sl])
                j0[k, sl] = (j0[k, sl] + 0.5 * j1[k, sl]
                             + (1.0 / 3.0) * j2[k, sl] + 0.25 * j3[k, sl])
            return carry2

        lax.fori_loop(0, _CH, comb, 0)
        b0 = (s * _CPT + jj) * _CH
        ou = pl.multiple_of((c * 3 + 0) * _B + b0, 8)
        oi = pl.multiple_of((c * 3 + 1) * _B + b0, 8)
        oj = pl.multiple_of((c * 3 + 2) * _B + b0, 8)
        pltpu.sync_copy(ub, out.at[pl.ds(ou, _CH)])
        pltpu.sync_copy(i0, out.at[pl.ds(oi, _CH)])
        pltpu.sync_copy(j0, out.at[pl.ds(oj, _CH)])
        return carry

    lax.fori_loop(0, _CPT, chunk, 0)


_gather = functools.partial(
    pl.kernel,
    out_type=jax.ShapeDtypeStruct((6 * _B, _D), jnp.float32),
    mesh=_mesh,
    scratch_types=(
        [pltpu.VMEM((_CPT, _CH), jnp.int32)] * 3
        + [pltpu.VMEM((_CH, _D), jnp.float32)] * 9
        + [pltpu.SemaphoreType.DMA]
    ),
    compiler_params=pltpu.CompilerParams(use_tc_tiling_on_sc=False,
                                         needs_layout_passes=False),
)(_gather_body)


_FB = 2048


def _fin_body(g_ref, du_ref, di_ref, out_ref):
    pid = pl.program_id(0)
    g = g_ref[...]
    u = g[:, 0]
    i = g[:, 1]
    j = g[:, 2]
    si = jnp.sum(u * i, axis=-1) / _T
    sj = jnp.sum(u * j, axis=-1) / _T
    num = jnp.exp(si)
    den = jnp.exp(sj) + num
    deg = jnp.stack([du_ref[...].reshape(_FB), di_ref[...].reshape(_FB)])
    part = -jnp.sum(jnp.log(num / den) * deg) / _N

    @pl.when(pid == 0)
    def _():
        out_ref[...] = jnp.zeros((1, 1), jnp.float32)

    out_ref[...] += jnp.reshape(part, (1, 1))


def kernel(user, item_i, item_j, degree_U, item_z_U, user_, item_i_, item_j_,
           degree_I, item_z_I, embed_user, embed_item, old_U_emb, old_I_emb,
           ui_rows, ui_cols, ui_vals):
    rows = ui_rows.astype(jnp.int32)
    cols = ui_cols.astype(jnp.int32)
    npad = _EPAD - _E
    pad_idx = jnp.arange(npad, dtype=jnp.int32) % _N
    pad_val = jnp.zeros((npad,), jnp.float32)
    cat = jnp.concatenate
    g_side0 = cat([cols + _N, pad_idx + _N])
    g_side1 = cat([rows, pad_idx])
    s_side0 = cat([rows, pad_idx])
    s_side1 = cat([cols, pad_idx])
    gidx = cat([g_side0, g_side1]).reshape(2 * _EROWS, _SUB)
    sidx = cat([s_side0, s_side1]).reshape(2 * _EROWS, _SUB)
    vals = cat([ui_vals.astype(jnp.float32), pad_val]).reshape(_EROWS, _SUB)

    e0 = cat([embed_user, embed_item])
    g1 = _spmm(e0, gidx, sidx, vals)
    g2 = _spmm(g1, gidx, sidx, vals)
    g3 = _spmm(g2, gidx, sidx, vals)

    olds = cat([old_U_emb, old_I_emb])
    i32 = jnp.int32

    def bidx(a, b):
        return cat([a.astype(i32), b.astype(i32) + _N]).reshape(
            2 * (_B // _CH), _CH)

    uidx = bidx(user, user_)
    iidx = bidx(item_i, item_i_)
    jidx = bidx(item_j, item_j_)
    gath = _gather(olds, e0, g1, g2, g3, uidx, iidx, jidx)

    nfb = _B // _FB
    dshape = (nfb, 16, _FB // 16)
    out = pl.pallas_call(
        _fin_body,
        grid=(nfb,),
        in_specs=[
            pl.BlockSpec((2, 3, _FB, _D), lambda b: (0, 0, b, 0)),
            pl.BlockSpec((1, 16, _FB // 16), lambda b: (b, 0, 0)),
            pl.BlockSpec((1, 16, _FB // 16), lambda b: (b, 0, 0)),
        ],
        out_specs=pl.BlockSpec((1, 1), lambda b: (0, 0)),
        out_shape=jax.ShapeDtypeStruct((1, 1), jnp.float32),
    )(gath.reshape(2, 3, _B, _D), degree_U.reshape(dshape),
      degree_I.reshape(dshape))
    return out.reshape(1)


# speedup vs baseline: 14.3263x; 1.0121x over previous
import functools

import jax
import jax.numpy as jnp
from jax import lax
from jax.experimental import pallas as pl
from jax.experimental.pallas import tpu as pltpu
from jax.experimental.pallas import tpu_sc as plsc

_N = 50000
_D = 32
_E = 1600000
_B = 16384
_T = 1.0

_SUB = 128
_NSUB = 3
_W = _SUB * _NSUB
_WINS = 268
_EPAD = 16 * _WINS * _W
_EROWS = _EPAD // _SUB

_RCH = 128
_NCHUNK = _N // _RCH
_NTAIL = _N - _NCHUNK * _RCH

_CH = 128
_CPT = _B // 16 // _CH

_mesh = plsc.VectorSubcoreMesh(core_axis_name="c", subcore_axis_name="s")


def _spmm_body(tabs, pk, out, acc, pkbuf, gbuf, sem):
    c = lax.axis_index("c")
    s = lax.axis_index("s")

    def zrow(k, carry):
        gbuf[0, 0, k, pl.ds(0, 16)] = jnp.zeros((16,), jnp.float32)
        gbuf[0, 0, k, pl.ds(16, 16)] = jnp.zeros((16,), jnp.float32)
        return carry

    lax.fori_loop(0, _SUB, zrow, 0)

    def zcp(i, carry):
        cid = s + 16 * i

        @pl.when(cid < _NCHUNK)
        def _():
            r = pl.multiple_of(cid * _RCH, 8)
            pltpu.sync_copy(gbuf.at[0, 0], acc.at[pl.ds(r, _RCH)])

        @pl.when(cid == _NCHUNK)
        def _():
            pltpu.sync_copy(gbuf.at[0, 0, pl.ds(0, _NTAIL)],
                            acc.at[pl.ds(_NCHUNK * _RCH, _NTAIL)])

        return carry

    lax.fori_loop(0, 25, zcp, 0)
    plsc.subcore_barrier()

    sem_g = (sem.at[0], sem.at[1])
    sem_s = (sem.at[2], sem.at[3])
    sem_i = (sem.at[4], sem.at[5])

    def idx_fire(w, sl):
        r0 = (c * 16 + s) * _WINS * _NSUB + w * _NSUB
        pltpu.async_copy(pk.at[pl.ds(r0, _NSUB)], pkbuf.at[sl], sem_i[sl])

    def idx_drain(sl):
        pltpu.make_async_copy(pk.at[pl.ds(0, _NSUB)], pkbuf.at[sl],
                              sem_i[sl]).wait()

    def gat_fire(sl):
        for j in range(_NSUB):
            pltpu.async_copy(tabs.at[pkbuf.at[sl, j, 0]], gbuf.at[sl, j],
                             sem_g[sl])

    def gat_drain(sl):
        for j in range(_NSUB):
            pltpu.make_async_copy(tabs.at[pkbuf.at[sl, j, 0]],
                                  gbuf.at[sl, j], sem_g[sl]).wait()

    def sca_fire(sl):
        for j in range(_NSUB):
            pltpu.async_copy(gbuf.at[sl, j], acc.at[pkbuf.at[sl, j, 1]],
                             sem_s[sl], add=True)

    def sca_drain(sl):
        for j in range(_NSUB):
            pltpu.make_async_copy(gbuf.at[sl, j], acc.at[pkbuf.at[sl, j, 1]],
                                  sem_s[sl]).wait()

    def scale(sl):
        for j in range(_NSUB):
            def sbody(k, carry2, j=j):
                kv = jnp.full((16,), 0, jnp.int32) + k
                vi = plsc.load_gather(pkbuf.at[sl, j, 2], [kv])
                v = plsc.bitcast(vi, jnp.float32)
                gbuf[sl, j, k, pl.ds(0, 16)] = gbuf[sl, j, k, pl.ds(0, 16)] * v
                gbuf[sl, j, k, pl.ds(16, 16)] = (
                    gbuf[sl, j, k, pl.ds(16, 16)] * v)
                return carry2

            lax.fori_loop(0, _SUB, sbody, 0, unroll=8)

    def process(w, cur, nxt):
        @pl.when(w + 1 < _WINS)
        def _():
            idx_fire(w + 1, nxt)

        @pl.when(w >= 1)
        def _():
            sca_drain(nxt)

        @pl.when(w + 1 < _WINS)
        def _():
            idx_drain(nxt)
            gat_fire(nxt)

        gat_drain(cur)
        scale(cur)
        sca_fire(cur)

    idx_fire(0, 0)
    idx_drain(0)
    gat_fire(0)

    def wpair(i, carry):
        process(2 * i, 0, 1)
        process(2 * i + 1, 1, 0)
        return carry

    lax.fori_loop(0, _WINS // 2, wpair, 0)
    sca_drain(1)
    plsc.subcore_barrier()

    def obody(i, carry):
        cid = s + 16 * i

        @pl.when(cid < _NCHUNK)
        def _():
            r = pl.multiple_of(cid * _RCH, 8)
            ro = pl.multiple_of(c * _N + cid * _RCH, 8)
            pltpu.sync_copy(acc.at[pl.ds(r, _RCH)], gbuf.at[0, 0])
            pltpu.sync_copy(gbuf.at[0, 0], out.at[pl.ds(ro, _RCH)])

        @pl.when(cid == _NCHUNK)
        def _():
            ro = pl.multiple_of(c * _N + _NCHUNK * _RCH, 8)
            pltpu.sync_copy(acc.at[pl.ds(_NCHUNK * _RCH, _NTAIL)],
                            gbuf.at[0, 0, pl.ds(0, _NTAIL)])
            pltpu.sync_copy(gbuf.at[0, 0, pl.ds(0, _NTAIL)],
                            out.at[pl.ds(ro, _NTAIL)])

        return carry

    lax.fori_loop(0, 25, obody, 0)


_spmm = functools.partial(
    pl.kernel,
    out_type=jax.ShapeDtypeStruct((2 * _N, _D), jnp.float32),
    mesh=_mesh,
    scratch_types=[
        pltpu.VMEM_SHARED((_N, _D), jnp.float32),
        pltpu.VMEM((2, _NSUB, 3, _SUB), jnp.int32),
        pltpu.VMEM((2, _NSUB, _SUB, _D), jnp.float32),
        pltpu.SemaphoreType.DMA((6,)),
    ],
    compiler_params=pltpu.CompilerParams(use_tc_tiling_on_sc=False,
                                         needs_layout_passes=False),
)(_spmm_body)


def _gather_body(olds, e0, g1, g2, g3, uidx, iidx, jidx, out,
                 ubi, ibi, jbi, ub, i0, i1, i2, i3, j0, j1, j2, j3, sem):
    c = lax.axis_index("c")
    s = lax.axis_index("s")
    r0 = pl.multiple_of(c * (_B // _CH) + s * _CPT, 8)
    pltpu.sync_copy(uidx.at[pl.ds(r0, _CPT)], ubi)
    pltpu.sync_copy(iidx.at[pl.ds(r0, _CPT)], ibi)
    pltpu.sync_copy(jidx.at[pl.ds(r0, _CPT)], jbi)

    def chunk(jj, carry):
        cps = [
            pltpu.async_copy(olds.at[ubi.at[jj]], ub, sem),
            pltpu.async_copy(e0.at[ibi.at[jj]], i0, sem),
            pltpu.async_copy(g1.at[ibi.at[jj]], i1, sem),
            pltpu.async_copy(g2.at[ibi.at[jj]], i2, sem),
            pltpu.async_copy(g3.at[ibi.at[jj]], i3, sem),
            pltpu.async_copy(e0.at[jbi.at[jj]], j0, sem),
            pltpu.async_copy(g1.at[jbi.at[jj]], j1, sem),
            pltpu.async_copy(g2.at[jbi.at[jj]], j2, sem),
            pltpu.async_copy(g3.at[jbi.at[jj]], j3, sem),
        ]
        for cp in cps:
            cp.wait()

        def comb(k, carry2):
            for sl in (pl.ds(0, 16), pl.ds(16, 16)):
                i0[k, sl] = (i0[k, sl] + 0.5 * i1[k, sl]
                             + (1.0 / 3.0) * i2[k, sl] + 0.25 * i3[k, sl])
                j0[k, sl] = (j0[k, sl] + 0.5 * j1[k, sl]
                             + (1.0 / 3.0) * j2[k, sl] + 0.25 * j3[k, sl])
            return carry2

        lax.fori_loop(0, _CH, comb, 0)
        b0 = (s * _CPT + jj) * _CH
        ou = pl.multiple_of((c * 3 + 0) * _B + b0, 8)
        oi = pl.multiple_of((c * 3 + 1) * _B + b0, 8)
        oj = pl.multiple_of((c * 3 + 2) * _B + b0, 8)
        pltpu.sync_copy(ub, out.at[pl.ds(ou, _CH)])
        pltpu.sync_copy(i0, out.at[pl.ds(oi, _CH)])
        pltpu.sync_copy(j0, out.at[pl.ds(oj, _CH)])
        return carry

    lax.fori_loop(0, _CPT, chunk, 0)


_gather = functools.partial(
    pl.kernel,
    out_type=jax.ShapeDtypeStruct((6 * _B, _D), jnp.float32),
    mesh=_mesh,
    scratch_types=(
        [pltpu.VMEM((_CPT, _CH), jnp.int32)] * 3
        + [pltpu.VMEM((_CH, _D), jnp.float32)] * 9
        + [pltpu.SemaphoreType.DMA]
    ),
    compiler_params=pltpu.CompilerParams(use_tc_tiling_on_sc=False,
                                         needs_layout_passes=False),
)(_gather_body)


_FB = 2048


def _fin_body(g_ref, du_ref, di_ref, out_ref):
    pid = pl.program_id(0)
    g = g_ref[...]
    u = g[:, 0]
    i = g[:, 1]
    j = g[:, 2]
    si = jnp.sum(u * i, axis=-1) / _T
    sj = jnp.sum(u * j, axis=-1) / _T
    num = jnp.exp(si)
    den = jnp.exp(sj) + num
    deg = jnp.stack([du_ref[...].reshape(_FB), di_ref[...].reshape(_FB)])
    part = -jnp.sum(jnp.log(num / den) * deg) / _N

    @pl.when(pid == 0)
    def _():
        out_ref[...] = jnp.zeros((1, 1), jnp.float32)

    out_ref[...] += jnp.reshape(part, (1, 1))


def kernel(user, item_i, item_j, degree_U, item_z_U, user_, item_i_, item_j_,
           degree_I, item_z_I, embed_user, embed_item, old_U_emb, old_I_emb,
           ui_rows, ui_cols, ui_vals):
    rows = ui_rows.astype(jnp.int32)
    cols = ui_cols.astype(jnp.int32)
    npad = _EPAD - _E
    pad_idx = jnp.arange(npad, dtype=jnp.int32) % _N
    pad_val = jnp.zeros((npad,), jnp.float32)
    cat = jnp.concatenate
    g_side0 = cat([cols + _N, pad_idx + _N]).reshape(_EROWS, _SUB)
    g_side1 = cat([rows, pad_idx]).reshape(_EROWS, _SUB)
    s_side0 = cat([rows, pad_idx]).reshape(_EROWS, _SUB)
    s_side1 = cat([cols, pad_idx]).reshape(_EROWS, _SUB)
    vals_i = lax.bitcast_convert_type(
        cat([ui_vals.astype(jnp.float32), pad_val]), jnp.int32
    ).reshape(_EROWS, _SUB)
    pk0 = jnp.stack([g_side0, s_side0, vals_i], axis=1)
    pk1 = jnp.stack([g_side1, s_side1, vals_i], axis=1)
    pk = cat([pk0, pk1])

    e0 = cat([embed_user, embed_item])
    g1 = _spmm(e0, pk)
    g2 = _spmm(g1, pk)
    g3 = _spmm(g2, pk)

    olds = cat([old_U_emb, old_I_emb])
    i32 = jnp.int32

    def bidx(a, b):
        return cat([a.astype(i32), b.astype(i32) + _N]).reshape(
            2 * (_B // _CH), _CH)

    uidx = bidx(user, user_)
    iidx = bidx(item_i, item_i_)
    jidx = bidx(item_j, item_j_)
    gath = _gather(olds, e0, g1, g2, g3, uidx, iidx, jidx)

    nfb = _B // _FB
    dshape = (nfb, 16, _FB // 16)
    out = pl.pallas_call(
        _fin_body,
        grid=(nfb,),
        in_specs=[
            pl.BlockSpec((2, 3, _FB, _D), lambda b: (0, 0, b, 0)),
            pl.BlockSpec((1, 16, _FB // 16), lambda b: (b, 0, 0)),
            pl.BlockSpec((1, 16, _FB // 16), lambda b: (b, 0, 0)),
        ],
        out_specs=pl.BlockSpec((1, 1), lambda b: (0, 0)),
        out_shape=jax.ShapeDtypeStruct((1, 1), jnp.float32),
    )(gath.reshape(2, 3, _B, _D), degree_U.reshape(dshape),
      degree_I.reshape(dshape))
    return out.reshape(1)


# speedup vs baseline: 26.6838x; 1.8626x over previous
import functools

import jax
import jax.numpy as jnp
from jax import lax
from jax.experimental import pallas as pl
from jax.experimental.pallas import tpu as pltpu
from jax.experimental.pallas import tpu_sc as plsc

_N = 50000
_D = 32
_E = 1600000
_B = 16384
_T = 1.0

_SUB = 128
_NSUB = 3
_W = _SUB * _NSUB
_WINS = 268
_EPAD = 16 * _WINS * _W
_EROWS = _EPAD // _SUB

_RCH = 128
_NCHUNK = _N // _RCH
_NTAIL = _N - _NCHUNK * _RCH

_CH = 128
_CPT = _B // 16 // _CH

_mesh = plsc.VectorSubcoreMesh(core_axis_name="c", subcore_axis_name="s")


def _spmm_body(tabs, pk, out, acc, pkbuf, gbuf, sem):
    c = lax.axis_index("c")
    s = lax.axis_index("s")

    def zrow(k, carry):
        gbuf[0, 0, k, pl.ds(0, 16)] = jnp.zeros((16,), jnp.float32)
        gbuf[0, 0, k, pl.ds(16, 16)] = jnp.zeros((16,), jnp.float32)
        return carry

    lax.fori_loop(0, _SUB, zrow, 0)

    def zcp(i, carry):
        cid = s + 16 * i

        @pl.when(cid < _NCHUNK)
        def _():
            r = pl.multiple_of(cid * _RCH, 8)
            pltpu.sync_copy(gbuf.at[0, 0], acc.at[pl.ds(r, _RCH)])

        @pl.when(cid == _NCHUNK)
        def _():
            pltpu.sync_copy(gbuf.at[0, 0, pl.ds(0, _NTAIL)],
                            acc.at[pl.ds(_NCHUNK * _RCH, _NTAIL)])

        return carry

    lax.fori_loop(0, 25, zcp, 0)
    plsc.subcore_barrier()

    sem_g = (sem.at[0], sem.at[1])
    sem_s = (sem.at[2], sem.at[3])
    sem_i = (sem.at[4], sem.at[5])

    def idx_fire(w, sl):
        r0 = (c * 16 + s) * _WINS * _NSUB + w * _NSUB
        pltpu.async_copy(pk.at[pl.ds(r0, _NSUB)], pkbuf.at[sl], sem_i[sl])

    def idx_drain(sl):
        pltpu.make_async_copy(pk.at[pl.ds(0, _NSUB)], pkbuf.at[sl],
                              sem_i[sl]).wait()

    def gat_fire(sl):
        for j in range(_NSUB):
            pltpu.async_copy(tabs.at[pkbuf.at[sl, j, 0]], gbuf.at[sl, j],
                             sem_g[sl])

    def gat_drain(sl):
        for j in range(_NSUB):
            pltpu.make_async_copy(tabs.at[pkbuf.at[sl, j, 0]],
                                  gbuf.at[sl, j], sem_g[sl]).wait()

    def sca_fire(sl):
        for j in range(_NSUB):
            pltpu.async_copy(gbuf.at[sl, j], acc.at[pkbuf.at[sl, j, 1]],
                             sem_s[sl], add=True)

    def sca_drain(sl):
        for j in range(_NSUB):
            pltpu.make_async_copy(gbuf.at[sl, j], acc.at[pkbuf.at[sl, j, 1]],
                                  sem_s[sl]).wait()

    def scale(sl):
        for j in range(_NSUB):
            def sbody(k16, carry2, j=j):
                base = k16 * 16
                vi = pkbuf[sl, j, 2, pl.ds(base, 16)]
                v16 = plsc.bitcast(vi, jnp.float32)
                for l in range(16):
                    vl = v16[l]
                    r = base + l
                    gbuf[sl, j, r, pl.ds(0, 16)] = (
                        gbuf[sl, j, r, pl.ds(0, 16)] * vl)
                    gbuf[sl, j, r, pl.ds(16, 16)] = (
                        gbuf[sl, j, r, pl.ds(16, 16)] * vl)
                return carry2

            lax.fori_loop(0, _SUB // 16, sbody, 0)

    def process(w, cur, nxt):
        @pl.when(w + 1 < _WINS)
        def _():
            idx_fire(w + 1, nxt)

        @pl.when(w >= 1)
        def _():
            sca_drain(nxt)

        @pl.when(w + 1 < _WINS)
        def _():
            idx_drain(nxt)
            gat_fire(nxt)

        gat_drain(cur)
        scale(cur)
        sca_fire(cur)

    idx_fire(0, 0)
    idx_drain(0)
    gat_fire(0)

    def wpair(i, carry):
        process(2 * i, 0, 1)
        process(2 * i + 1, 1, 0)
        return carry

    lax.fori_loop(0, _WINS // 2, wpair, 0)
    sca_drain(1)
    plsc.subcore_barrier()

    def obody(i, carry):
        cid = s + 16 * i

        @pl.when(cid < _NCHUNK)
        def _():
            r = pl.multiple_of(cid * _RCH, 8)
            ro = pl.multiple_of(c * _N + cid * _RCH, 8)
            pltpu.sync_copy(acc.at[pl.ds(r, _RCH)], gbuf.at[0, 0])
            pltpu.sync_copy(gbuf.at[0, 0], out.at[pl.ds(ro, _RCH)])

        @pl.when(cid == _NCHUNK)
        def _():
            ro = pl.multiple_of(c * _N + _NCHUNK * _RCH, 8)
            pltpu.sync_copy(acc.at[pl.ds(_NCHUNK * _RCH, _NTAIL)],
                            gbuf.at[0, 0, pl.ds(0, _NTAIL)])
            pltpu.sync_copy(gbuf.at[0, 0, pl.ds(0, _NTAIL)],
                            out.at[pl.ds(ro, _NTAIL)])

        return carry

    lax.fori_loop(0, 25, obody, 0)


_spmm = functools.partial(
    pl.kernel,
    out_type=jax.ShapeDtypeStruct((2 * _N, _D), jnp.float32),
    mesh=_mesh,
    scratch_types=[
        pltpu.VMEM_SHARED((_N, _D), jnp.float32),
        pltpu.VMEM((2, _NSUB, 3, _SUB), jnp.int32),
        pltpu.VMEM((2, _NSUB, _SUB, _D), jnp.float32),
        pltpu.SemaphoreType.DMA((6,)),
    ],
    compiler_params=pltpu.CompilerParams(use_tc_tiling_on_sc=False,
                                         needs_layout_passes=False),
)(_spmm_body)


def _gather_body(olds, e0, g1, g2, g3, uidx, iidx, jidx, out,
                 ubi, ibi, jbi, ub, i0, i1, i2, i3, j0, j1, j2, j3, sem):
    c = lax.axis_index("c")
    s = lax.axis_index("s")
    r0 = pl.multiple_of(c * (_B // _CH) + s * _CPT, 8)
    pltpu.sync_copy(uidx.at[pl.ds(r0, _CPT)], ubi)
    pltpu.sync_copy(iidx.at[pl.ds(r0, _CPT)], ibi)
    pltpu.sync_copy(jidx.at[pl.ds(r0, _CPT)], jbi)

    def chunk(jj, carry):
        cps = [
            pltpu.async_copy(olds.at[ubi.at[jj]], ub, sem),
            pltpu.async_copy(e0.at[ibi.at[jj]], i0, sem),
            pltpu.async_copy(g1.at[ibi.at[jj]], i1, sem),
            pltpu.async_copy(g2.at[ibi.at[jj]], i2, sem),
            pltpu.async_copy(g3.at[ibi.at[jj]], i3, sem),
            pltpu.async_copy(e0.at[jbi.at[jj]], j0, sem),
            pltpu.async_copy(g1.at[jbi.at[jj]], j1, sem),
            pltpu.async_copy(g2.at[jbi.at[jj]], j2, sem),
            pltpu.async_copy(g3.at[jbi.at[jj]], j3, sem),
        ]
        for cp in cps:
            cp.wait()

        def comb(k, carry2):
            for sl in (pl.ds(0, 16), pl.ds(16, 16)):
                i0[k, sl] = (i0[k, sl] + 0.5 * i1[k, sl]
                             + (1.0 / 3.0) * i2[k, sl] + 0.25 * i3[k, sl])
                j0[k, sl] = (j0[k, sl] + 0.5 * j1[k, sl]
                             + (1.0 / 3.0) * j2[k, sl] + 0.25 * j3[k, sl])
            return carry2

        lax.fori_loop(0, _CH, comb, 0)
        b0 = (s * _CPT + jj) * _CH
        ou = pl.multiple_of((c * 3 + 0) * _B + b0, 8)
        oi = pl.multiple_of((c * 3 + 1) * _B + b0, 8)
        oj = pl.multiple_of((c * 3 + 2) * _B + b0, 8)
        pltpu.sync_copy(ub, out.at[pl.ds(ou, _CH)])
        pltpu.sync_copy(i0, out.at[pl.ds(oi, _CH)])
        pltpu.sync_copy(j0, out.at[pl.ds(oj, _CH)])
        return carry

    lax.fori_loop(0, _CPT, chunk, 0)


_gather = functools.partial(
    pl.kernel,
    out_type=jax.ShapeDtypeStruct((6 * _B, _D), jnp.float32),
    mesh=_mesh,
    scratch_types=(
        [pltpu.VMEM((_CPT, _CH), jnp.int32)] * 3
        + [pltpu.VMEM((_CH, _D), jnp.float32)] * 9
        + [pltpu.SemaphoreType.DMA]
    ),
    compiler_params=pltpu.CompilerParams(use_tc_tiling_on_sc=False,
                                         needs_layout_passes=False),
)(_gather_body)


_FB = 2048


def _fin_body(g_ref, du_ref, di_ref, out_ref):
    pid = pl.program_id(0)
    g = g_ref[...]
    u = g[:, 0]
    i = g[:, 1]
    j = g[:, 2]
    si = jnp.sum(u * i, axis=-1) / _T
    sj = jnp.sum(u * j, axis=-1) / _T
    num = jnp.exp(si)
    den = jnp.exp(sj) + num
    deg = jnp.stack([du_ref[...].reshape(_FB), di_ref[...].reshape(_FB)])
    part = -jnp.sum(jnp.log(num / den) * deg) / _N

    @pl.when(pid == 0)
    def _():
        out_ref[...] = jnp.zeros((1, 1), jnp.float32)

    out_ref[...] += jnp.reshape(part, (1, 1))


def kernel(user, item_i, item_j, degree_U, item_z_U, user_, item_i_, item_j_,
           degree_I, item_z_I, embed_user, embed_item, old_U_emb, old_I_emb,
           ui_rows, ui_cols, ui_vals):
    rows = ui_rows.astype(jnp.int32)
    cols = ui_cols.astype(jnp.int32)
    npad = _EPAD - _E
    pad_idx = jnp.arange(npad, dtype=jnp.int32) % _N
    pad_val = jnp.zeros((npad,), jnp.float32)
    cat = jnp.concatenate
    g_side0 = cat([cols + _N, pad_idx + _N]).reshape(_EROWS, _SUB)
    g_side1 = cat([rows, pad_idx]).reshape(_EROWS, _SUB)
    s_side0 = cat([rows, pad_idx]).reshape(_EROWS, _SUB)
    s_side1 = cat([cols, pad_idx]).reshape(_EROWS, _SUB)
    vals_i = lax.bitcast_convert_type(
        cat([ui_vals.astype(jnp.float32), pad_val]), jnp.int32
    ).reshape(_EROWS, _SUB)
    pk0 = jnp.stack([g_side0, s_side0, vals_i], axis=1)
    pk1 = jnp.stack([g_side1, s_side1, vals_i], axis=1)
    pk = cat([pk0, pk1])

    e0 = cat([embed_user, embed_item])
    g1 = _spmm(e0, pk)
    g2 = _spmm(g1, pk)
    g3 = _spmm(g2, pk)

    olds = cat([old_U_emb, old_I_emb])
    i32 = jnp.int32

    def bidx(a, b):
        return cat([a.astype(i32), b.astype(i32) + _N]).reshape(
            2 * (_B // _CH), _CH)

    uidx = bidx(user, user_)
    iidx = bidx(item_i, item_i_)
    jidx = bidx(item_j, item_j_)
    gath = _gather(olds, e0, g1, g2, g3, uidx, iidx, jidx)

    nfb = _B // _FB
    dshape = (nfb, 16, _FB // 16)
    out = pl.pallas_call(
        _fin_body,
        grid=(nfb,),
        in_specs=[
            pl.BlockSpec((2, 3, _FB, _D), lambda b: (0, 0, b, 0)),
            pl.BlockSpec((1, 16, _FB // 16), lambda b: (b, 0, 0)),
            pl.BlockSpec((1, 16, _FB // 16), lambda b: (b, 0, 0)),
        ],
        out_specs=pl.BlockSpec((1, 1), lambda b: (0, 0)),
        out_shape=jax.ShapeDtypeStruct((1, 1), jnp.float32),
    )(gath.reshape(2, 3, _B, _D), degree_U.reshape(dshape),
      degree_I.reshape(dshape))
    return out.reshape(1)


# speedup vs baseline: 26.7316x; 1.0018x over previous
import functools

import jax
import jax.numpy as jnp
from jax import lax
from jax.experimental import pallas as pl
from jax.experimental.pallas import tpu as pltpu
from jax.experimental.pallas import tpu_sc as plsc

_N = 50000
_D = 32
_E = 1600000
_B = 16384
_T = 1.0

_SUB = 128
_NSUB = 3
_W = _SUB * _NSUB
_WINS = 268
_EPAD = 16 * _WINS * _W
_EROWS = _EPAD // _SUB

_RCH = 128
_NCHUNK = _N // _RCH
_NTAIL = _N - _NCHUNK * _RCH

_CH = 128
_CPT = _B // 16 // _CH

_mesh = plsc.VectorSubcoreMesh(core_axis_name="c", subcore_axis_name="s")


def _spmm_body(tabs, pk, out, acc, pkbuf, gbuf, sem):
    c = lax.axis_index("c")
    s = lax.axis_index("s")

    def zrow(k, carry):
        gbuf[0, 0, k, pl.ds(0, 16)] = jnp.zeros((16,), jnp.float32)
        gbuf[0, 0, k, pl.ds(16, 16)] = jnp.zeros((16,), jnp.float32)
        return carry

    lax.fori_loop(0, _SUB, zrow, 0)

    def zcp(i, carry):
        cid = s + 16 * i

        @pl.when(cid < _NCHUNK)
        def _():
            r = pl.multiple_of(cid * _RCH, 8)
            pltpu.sync_copy(gbuf.at[0, 0], acc.at[pl.ds(r, _RCH)])

        @pl.when(cid == _NCHUNK)
        def _():
            pltpu.sync_copy(gbuf.at[0, 0, pl.ds(0, _NTAIL)],
                            acc.at[pl.ds(_NCHUNK * _RCH, _NTAIL)])

        return carry

    lax.fori_loop(0, 25, zcp, 0)
    plsc.subcore_barrier()

    sem_g = (sem.at[0], sem.at[1])
    sem_s = (sem.at[2], sem.at[3])
    sem_i = (sem.at[4], sem.at[5])

    def idx_fire(w, sl):
        r0 = (c * 16 + s) * _WINS * _NSUB + w * _NSUB
        pltpu.async_copy(pk.at[pl.ds(r0, _NSUB)], pkbuf.at[sl], sem_i[sl])

    def idx_drain(sl):
        pltpu.make_async_copy(pk.at[pl.ds(0, _NSUB)], pkbuf.at[sl],
                              sem_i[sl]).wait()

    def gat_fire(sl):
        for j in range(_NSUB):
            pltpu.async_copy(tabs.at[pkbuf.at[sl, j, 0]], gbuf.at[sl, j],
                             sem_g[sl])

    def gat_drain(sl):
        for j in range(_NSUB):
            pltpu.make_async_copy(tabs.at[pkbuf.at[sl, j, 0]],
                                  gbuf.at[sl, j], sem_g[sl]).wait()

    def sca_fire(sl):
        for j in range(_NSUB):
            pltpu.async_copy(gbuf.at[sl, j], acc.at[pkbuf.at[sl, j, 1]],
                             sem_s[sl], add=True)

    def sca_drain(sl):
        for j in range(_NSUB):
            pltpu.make_async_copy(gbuf.at[sl, j], acc.at[pkbuf.at[sl, j, 1]],
                                  sem_s[sl]).wait()

    def scale(sl):
        for j in range(_NSUB):
            def sbody(k16, carry2, j=j):
                base = k16 * 16
                vi = pkbuf[sl, j, 2, pl.ds(base, 16)]
                v16 = plsc.bitcast(vi, jnp.float32)
                for g in range(2):
                    rows = [
                        (gbuf[sl, j, base + g * 8 + l, pl.ds(0, 16)],
                         gbuf[sl, j, base + g * 8 + l, pl.ds(16, 16)])
                        for l in range(8)
                    ]
                    for l in range(8):
                        vl = v16[g * 8 + l]
                        r = base + g * 8 + l
                        gbuf[sl, j, r, pl.ds(0, 16)] = rows[l][0] * vl
                        gbuf[sl, j, r, pl.ds(16, 16)] = rows[l][1] * vl
                return carry2

            lax.fori_loop(0, _SUB // 16, sbody, 0)

    def process(w, cur, nxt):
        @pl.when(w + 1 < _WINS)
        def _():
            idx_fire(w + 1, nxt)

        @pl.when(w >= 1)
        def _():
            sca_drain(nxt)

        @pl.when(w + 1 < _WINS)
        def _():
            idx_drain(nxt)
            gat_fire(nxt)

        gat_drain(cur)
        scale(cur)
        sca_fire(cur)

    idx_fire(0, 0)
    idx_drain(0)
    gat_fire(0)

    def wpair(i, carry):
        process(2 * i, 0, 1)
        process(2 * i + 1, 1, 0)
        return carry

    lax.fori_loop(0, _WINS // 2, wpair, 0)
    sca_drain(1)
    plsc.subcore_barrier()

    def obody(i, carry):
        cid = s + 16 * i

        @pl.when(cid < _NCHUNK)
        def _():
            r = pl.multiple_of(cid * _RCH, 8)
            ro = pl.multiple_of(c * _N + cid * _RCH, 8)
            pltpu.sync_copy(acc.at[pl.ds(r, _RCH)], gbuf.at[0, 0])
            pltpu.sync_copy(gbuf.at[0, 0], out.at[pl.ds(ro, _RCH)])

        @pl.when(cid == _NCHUNK)
        def _():
            ro = pl.multiple_of(c * _N + _NCHUNK * _RCH, 8)
            pltpu.sync_copy(acc.at[pl.ds(_NCHUNK * _RCH, _NTAIL)],
                            gbuf.at[0, 0, pl.ds(0, _NTAIL)])
            pltpu.sync_copy(gbuf.at[0, 0, pl.ds(0, _NTAIL)],
                            out.at[pl.ds(ro, _NTAIL)])

        return carry

    lax.fori_loop(0, 25, obody, 0)


_spmm = functools.partial(
    pl.kernel,
    out_type=jax.ShapeDtypeStruct((2 * _N, _D), jnp.float32),
    mesh=_mesh,
    scratch_types=[
        pltpu.VMEM_SHARED((_N, _D), jnp.float32),
        pltpu.VMEM((2, _NSUB, 3, _SUB), jnp.int32),
        pltpu.VMEM((2, _NSUB, _SUB, _D), jnp.float32),
        pltpu.SemaphoreType.DMA((6,)),
    ],
    compiler_params=pltpu.CompilerParams(use_tc_tiling_on_sc=False,
                                         needs_layout_passes=False),
)(_spmm_body)


def _gather_body(olds, e0, g1, g2, g3, uidx, iidx, jidx, out,
                 ubi, ibi, jbi, ub, i0, i1, i2, i3, j0, j1, j2, j3, sem):
    c = lax.axis_index("c")
    s = lax.axis_index("s")
    r0 = pl.multiple_of(c * (_B // _CH) + s * _CPT, 8)
    pltpu.sync_copy(uidx.at[pl.ds(r0, _CPT)], ubi)
    pltpu.sync_copy(iidx.at[pl.ds(r0, _CPT)], ibi)
    pltpu.sync_copy(jidx.at[pl.ds(r0, _CPT)], jbi)

    def chunk(jj, carry):
        cps = [
            pltpu.async_copy(olds.at[ubi.at[jj]], ub, sem),
            pltpu.async_copy(e0.at[ibi.at[jj]], i0, sem),
            pltpu.async_copy(g1.at[ibi.at[jj]], i1, sem),
            pltpu.async_copy(g2.at[ibi.at[jj]], i2, sem),
            pltpu.async_copy(g3.at[ibi.at[jj]], i3, sem),
            pltpu.async_copy(e0.at[jbi.at[jj]], j0, sem),
            pltpu.async_copy(g1.at[jbi.at[jj]], j1, sem),
            pltpu.async_copy(g2.at[jbi.at[jj]], j2, sem),
            pltpu.async_copy(g3.at[jbi.at[jj]], j3, sem),
        ]
        for cp in cps:
            cp.wait()

        def comb(k, carry2):
            for sl in (pl.ds(0, 16), pl.ds(16, 16)):
                i0[k, sl] = (i0[k, sl] + 0.5 * i1[k, sl]
                             + (1.0 / 3.0) * i2[k, sl] + 0.25 * i3[k, sl])
                j0[k, sl] = (j0[k, sl] + 0.5 * j1[k, sl]
                             + (1.0 / 3.0) * j2[k, sl] + 0.25 * j3[k, sl])
            return carry2

        lax.fori_loop(0, _CH, comb, 0)
        b0 = (s * _CPT + jj) * _CH
        ou = pl.multiple_of((c * 3 + 0) * _B + b0, 8)
        oi = pl.multiple_of((c * 3 + 1) * _B + b0, 8)
        oj = pl.multiple_of((c * 3 + 2) * _B + b0, 8)
        pltpu.sync_copy(ub, out.at[pl.ds(ou, _CH)])
        pltpu.sync_copy(i0, out.at[pl.ds(oi, _CH)])
        pltpu.sync_copy(j0, out.at[pl.ds(oj, _CH)])
        return carry

    lax.fori_loop(0, _CPT, chunk, 0)


_gather = functools.partial(
    pl.kernel,
    out_type=jax.ShapeDtypeStruct((6 * _B, _D), jnp.float32),
    mesh=_mesh,
    scratch_types=(
        [pltpu.VMEM((_CPT, _CH), jnp.int32)] * 3
        + [pltpu.VMEM((_CH, _D), jnp.float32)] * 9
        + [pltpu.SemaphoreType.DMA]
    ),
    compiler_params=pltpu.CompilerParams(use_tc_tiling_on_sc=False,
                                         needs_layout_passes=False),
)(_gather_body)


_FB = 2048


def _fin_body(g_ref, du_ref, di_ref, out_ref):
    pid = pl.program_id(0)
    g = g_ref[...]
    u = g[:, 0]
    i = g[:, 1]
    j = g[:, 2]
    si = jnp.sum(u * i, axis=-1) / _T
    sj = jnp.sum(u * j, axis=-1) / _T
    num = jnp.exp(si)
    den = jnp.exp(sj) + num
    deg = jnp.stack([du_ref[...].reshape(_FB), di_ref[...].reshape(_FB)])
    part = -jnp.sum(jnp.log(num / den) * deg) / _N

    @pl.when(pid == 0)
    def _():
        out_ref[...] = jnp.zeros((1, 1), jnp.float32)

    out_ref[...] += jnp.reshape(part, (1, 1))


def kernel(user, item_i, item_j, degree_U, item_z_U, user_, item_i_, item_j_,
           degree_I, item_z_I, embed_user, embed_item, old_U_emb, old_I_emb,
           ui_rows, ui_cols, ui_vals):
    rows = ui_rows.astype(jnp.int32)
    cols = ui_cols.astype(jnp.int32)
    npad = _EPAD - _E
    pad_idx = jnp.arange(npad, dtype=jnp.int32) % _N
    pad_val = jnp.zeros((npad,), jnp.float32)
    cat = jnp.concatenate
    g_side0 = cat([cols + _N, pad_idx + _N]).reshape(_EROWS, _SUB)
    g_side1 = cat([rows, pad_idx]).reshape(_EROWS, _SUB)
    s_side0 = cat([rows, pad_idx]).reshape(_EROWS, _SUB)
    s_side1 = cat([cols, pad_idx]).reshape(_EROWS, _SUB)
    vals_i = lax.bitcast_convert_type(
        cat([ui_vals.astype(jnp.float32), pad_val]), jnp.int32
    ).reshape(_EROWS, _SUB)
    pk0 = jnp.stack([g_side0, s_side0, vals_i], axis=1)
    pk1 = jnp.stack([g_side1, s_side1, vals_i], axis=1)
    pk = cat([pk0, pk1])

    e0 = cat([embed_user, embed_item])
    g1 = _spmm(e0, pk)
    g2 = _spmm(g1, pk)
    g3 = _spmm(g2, pk)

    olds = cat([old_U_emb, old_I_emb])
    i32 = jnp.int32

    def bidx(a, b):
        return cat([a.astype(i32), b.astype(i32) + _N]).reshape(
            2 * (_B // _CH), _CH)

    uidx = bidx(user, user_)
    iidx = bidx(item_i, item_i_)
    jidx = bidx(item_j, item_j_)
    gath = _gather(olds, e0, g1, g2, g3, uidx, iidx, jidx)

    nfb = _B // _FB
    dshape = (nfb, 16, _FB // 16)
    out = pl.pallas_call(
        _fin_body,
        grid=(nfb,),
        in_specs=[
            pl.BlockSpec((2, 3, _FB, _D), lambda b: (0, 0, b, 0)),
            pl.BlockSpec((1, 16, _FB // 16), lambda b: (b, 0, 0)),
            pl.BlockSpec((1, 16, _FB // 16), lambda b: (b, 0, 0)),
        ],
        out_specs=pl.BlockSpec((1, 1), lambda b: (0, 0)),
        out_shape=jax.ShapeDtypeStruct((1, 1), jnp.float32),
    )(gath.reshape(2, 3, _B, _D), degree_U.reshape(dshape),
      degree_I.reshape(dshape))
    return out.reshape(1)


# speedup vs baseline: 29.8745x; 1.1176x over previous
import functools

import jax
import jax.numpy as jnp
from jax import lax
from jax.experimental import pallas as pl
from jax.experimental.pallas import tpu as pltpu
from jax.experimental.pallas import tpu_sc as plsc

_N = 50000
_D = 32
_E = 1600000
_B = 16384
_T = 1.0

_SUB = 128
_NSUB = 3
_W = _SUB * _NSUB
_WINS = 268
_EPAD = 16 * _WINS * _W
_EROWS = _EPAD // _SUB

_RCH = 128
_NCHUNK = _N // _RCH
_NTAIL = _N - _NCHUNK * _RCH

_CH = 128
_CPT = _B // 16 // _CH

_mesh = plsc.VectorSubcoreMesh(core_axis_name="c", subcore_axis_name="s")


def _spmm_body(tabs, gidx, sidx, vals, out, acc, cbuf, sbuf, vbuf, gbuf,
               sem):
    c = lax.axis_index("c")
    s = lax.axis_index("s")

    def zrow(k, carry):
        gbuf[0, 0, k, pl.ds(0, 16)] = jnp.zeros((16,), jnp.float32)
        gbuf[0, 0, k, pl.ds(16, 16)] = jnp.zeros((16,), jnp.float32)
        return carry

    lax.fori_loop(0, _SUB, zrow, 0)

    def zcp(i, carry):
        cid = s + 16 * i

        @pl.when(cid < _NCHUNK)
        def _():
            r = pl.multiple_of(cid * _RCH, 8)
            pltpu.sync_copy(gbuf.at[0, 0], acc.at[pl.ds(r, _RCH)])

        @pl.when(cid == _NCHUNK)
        def _():
            pltpu.sync_copy(gbuf.at[0, 0, pl.ds(0, _NTAIL)],
                            acc.at[pl.ds(_NCHUNK * _RCH, _NTAIL)])

        return carry

    lax.fori_loop(0, 25, zcp, 0)
    plsc.subcore_barrier()

    sem_g = (sem.at[0], sem.at[1])
    sem_s = (sem.at[2], sem.at[3])
    sem_i = (sem.at[4], sem.at[5])

    def idx_fire(w, sl):
        r0 = (c * 16 + s) * _WINS * _NSUB + w * _NSUB
        rv = s * _WINS * _NSUB + w * _NSUB
        pltpu.async_copy(gidx.at[pl.ds(r0, _NSUB)], cbuf.at[sl], sem_i[sl])
        pltpu.async_copy(sidx.at[pl.ds(r0, _NSUB)], sbuf.at[sl], sem_i[sl])
        pltpu.async_copy(vals.at[pl.ds(rv, _NSUB)], vbuf.at[sl], sem_i[sl])

    def idx_drain(sl):
        pltpu.make_async_copy(gidx.at[pl.ds(0, _NSUB)], cbuf.at[sl],
                              sem_i[sl]).wait()
        pltpu.make_async_copy(sidx.at[pl.ds(0, _NSUB)], sbuf.at[sl],
                              sem_i[sl]).wait()
        pltpu.make_async_copy(vals.at[pl.ds(0, _NSUB)], vbuf.at[sl],
                              sem_i[sl]).wait()

    def gat_fire(sl):
        for j in range(_NSUB):
            pltpu.async_copy(tabs.at[cbuf.at[sl, j]], gbuf.at[sl, j],
                             sem_g[sl])

    def gat_drain(sl):
        for j in range(_NSUB):
            pltpu.make_async_copy(tabs.at[cbuf.at[sl, j]],
                                  gbuf.at[sl, j], sem_g[sl]).wait()

    def sca_fire(sl):
        for j in range(_NSUB):
            pltpu.async_copy(gbuf.at[sl, j], acc.at[sbuf.at[sl, j]],
                             sem_s[sl], add=True)

    def sca_drain(sl):
        for j in range(_NSUB):
            pltpu.make_async_copy(gbuf.at[sl, j], acc.at[sbuf.at[sl, j]],
                                  sem_s[sl]).wait()

    def scale(sl):
        for j in range(_NSUB):
            def sbody(k16, carry2, j=j):
                base = k16 * 16
                v16 = vbuf[sl, j, pl.ds(base, 16)]
                for g in range(2):
                    rows = [
                        (gbuf[sl, j, base + g * 8 + l, pl.ds(0, 16)],
                         gbuf[sl, j, base + g * 8 + l, pl.ds(16, 16)])
                        for l in range(8)
                    ]
                    for l in range(8):
                        vl = v16[g * 8 + l]
                        r = base + g * 8 + l
                        gbuf[sl, j, r, pl.ds(0, 16)] = rows[l][0] * vl
                        gbuf[sl, j, r, pl.ds(16, 16)] = rows[l][1] * vl
                return carry2

            lax.fori_loop(0, _SUB // 16, sbody, 0)

    def process(w, cur, nxt):
        @pl.when(w + 1 < _WINS)
        def _():
            idx_fire(w + 1, nxt)

        @pl.when(w >= 1)
        def _():
            sca_drain(nxt)

        @pl.when(w + 1 < _WINS)
        def _():
            idx_drain(nxt)
            gat_fire(nxt)

        gat_drain(cur)
        scale(cur)
        sca_fire(cur)

    idx_fire(0, 0)
    idx_drain(0)
    gat_fire(0)

    def wpair(i, carry):
        process(2 * i, 0, 1)
        process(2 * i + 1, 1, 0)
        return carry

    lax.fori_loop(0, _WINS // 2, wpair, 0)
    sca_drain(1)
    plsc.subcore_barrier()

    def obody(i, carry):
        cid = s + 16 * i

        @pl.when(cid < _NCHUNK)
        def _():
            r = pl.multiple_of(cid * _RCH, 8)
            ro = pl.multiple_of(c * _N + cid * _RCH, 8)
            pltpu.sync_copy(acc.at[pl.ds(r, _RCH)], gbuf.at[0, 0])
            pltpu.sync_copy(gbuf.at[0, 0], out.at[pl.ds(ro, _RCH)])

        @pl.when(cid == _NCHUNK)
        def _():
            ro = pl.multiple_of(c * _N + _NCHUNK * _RCH, 8)
            pltpu.sync_copy(acc.at[pl.ds(_NCHUNK * _RCH, _NTAIL)],
                            gbuf.at[0, 0, pl.ds(0, _NTAIL)])
            pltpu.sync_copy(gbuf.at[0, 0, pl.ds(0, _NTAIL)],
                            out.at[pl.ds(ro, _NTAIL)])

        return carry

    lax.fori_loop(0, 25, obody, 0)


_spmm = functools.partial(
    pl.kernel,
    out_type=jax.ShapeDtypeStruct((2 * _N, _D), jnp.float32),
    mesh=_mesh,
    scratch_types=[
        pltpu.VMEM_SHARED((_N, _D), jnp.float32),
        pltpu.VMEM((2, _NSUB, _SUB), jnp.int32),
        pltpu.VMEM((2, _NSUB, _SUB), jnp.int32),
        pltpu.VMEM((2, _NSUB, _SUB), jnp.float32),
        pltpu.VMEM((2, _NSUB, _SUB, _D), jnp.float32),
        pltpu.SemaphoreType.DMA((6,)),
    ],
    compiler_params=pltpu.CompilerParams(use_tc_tiling_on_sc=False,
                                         needs_layout_passes=False),
)(_spmm_body)


def _gather_body(olds, e0, g1, g2, g3, uidx, iidx, jidx, out,
                 ubi, ibi, jbi, ub, i0, i1, i2, i3, j0, j1, j2, j3, sem):
    c = lax.axis_index("c")
    s = lax.axis_index("s")
    r0 = pl.multiple_of(c * (_B // _CH) + s * _CPT, 8)
    pltpu.sync_copy(uidx.at[pl.ds(r0, _CPT)], ubi)
    pltpu.sync_copy(iidx.at[pl.ds(r0, _CPT)], ibi)
    pltpu.sync_copy(jidx.at[pl.ds(r0, _CPT)], jbi)

    def chunk(jj, carry):
        cps = [
            pltpu.async_copy(olds.at[ubi.at[jj]], ub, sem),
            pltpu.async_copy(e0.at[ibi.at[jj]], i0, sem),
            pltpu.async_copy(g1.at[ibi.at[jj]], i1, sem),
            pltpu.async_copy(g2.at[ibi.at[jj]], i2, sem),
            pltpu.async_copy(g3.at[ibi.at[jj]], i3, sem),
            pltpu.async_copy(e0.at[jbi.at[jj]], j0, sem),
            pltpu.async_copy(g1.at[jbi.at[jj]], j1, sem),
            pltpu.async_copy(g2.at[jbi.at[jj]], j2, sem),
            pltpu.async_copy(g3.at[jbi.at[jj]], j3, sem),
        ]
        for cp in cps:
            cp.wait()

        def comb(k, carry2):
            for sl in (pl.ds(0, 16), pl.ds(16, 16)):
                i0[k, sl] = (i0[k, sl] + 0.5 * i1[k, sl]
                             + (1.0 / 3.0) * i2[k, sl] + 0.25 * i3[k, sl])
                j0[k, sl] = (j0[k, sl] + 0.5 * j1[k, sl]
                             + (1.0 / 3.0) * j2[k, sl] + 0.25 * j3[k, sl])
            return carry2

        lax.fori_loop(0, _CH, comb, 0)
        b0 = (s * _CPT + jj) * _CH
        ou = pl.multiple_of((c * 3 + 0) * _B + b0, 8)
        oi = pl.multiple_of((c * 3 + 1) * _B + b0, 8)
        oj = pl.multiple_of((c * 3 + 2) * _B + b0, 8)
        pltpu.sync_copy(ub, out.at[pl.ds(ou, _CH)])
        pltpu.sync_copy(i0, out.at[pl.ds(oi, _CH)])
        pltpu.sync_copy(j0, out.at[pl.ds(oj, _CH)])
        return carry

    lax.fori_loop(0, _CPT, chunk, 0)


_gather = functools.partial(
    pl.kernel,
    out_type=jax.ShapeDtypeStruct((6 * _B, _D), jnp.float32),
    mesh=_mesh,
    scratch_types=(
        [pltpu.VMEM((_CPT, _CH), jnp.int32)] * 3
        + [pltpu.VMEM((_CH, _D), jnp.float32)] * 9
        + [pltpu.SemaphoreType.DMA]
    ),
    compiler_params=pltpu.CompilerParams(use_tc_tiling_on_sc=False,
                                         needs_layout_passes=False),
)(_gather_body)


_FB = 2048


def _fin_body(g_ref, du_ref, di_ref, out_ref):
    pid = pl.program_id(0)
    g = g_ref[...]
    u = g[:, 0]
    i = g[:, 1]
    j = g[:, 2]
    si = jnp.sum(u * i, axis=-1) / _T
    sj = jnp.sum(u * j, axis=-1) / _T
    num = jnp.exp(si)
    den = jnp.exp(sj) + num
    deg = jnp.stack([du_ref[...].reshape(_FB), di_ref[...].reshape(_FB)])
    part = -jnp.sum(jnp.log(num / den) * deg) / _N

    @pl.when(pid == 0)
    def _():
        out_ref[...] = jnp.zeros((1, 1), jnp.float32)

    out_ref[...] += jnp.reshape(part, (1, 1))


def kernel(user, item_i, item_j, degree_U, item_z_U, user_, item_i_, item_j_,
           degree_I, item_z_I, embed_user, embed_item, old_U_emb, old_I_emb,
           ui_rows, ui_cols, ui_vals):
    rows = ui_rows.astype(jnp.int32)
    cols = ui_cols.astype(jnp.int32)
    npad = _EPAD - _E
    pad_idx = jnp.arange(npad, dtype=jnp.int32) % _N
    pad_val = jnp.zeros((npad,), jnp.float32)
    cat = jnp.concatenate
    gidx = cat([cols + _N, pad_idx + _N, rows, pad_idx]).reshape(
        2 * _EROWS, _SUB)
    sidx = cat([rows, pad_idx, cols, pad_idx]).reshape(2 * _EROWS, _SUB)
    vals = cat([ui_vals.astype(jnp.float32), pad_val]).reshape(_EROWS, _SUB)

    e0 = cat([embed_user, embed_item])
    g1 = _spmm(e0, gidx, sidx, vals)
    g2 = _spmm(g1, gidx, sidx, vals)
    g3 = _spmm(g2, gidx, sidx, vals)

    olds = cat([old_U_emb, old_I_emb])
    i32 = jnp.int32

    def bidx(a, b):
        return cat([a.astype(i32), b.astype(i32) + _N]).reshape(
            2 * (_B // _CH), _CH)

    uidx = bidx(user, user_)
    iidx = bidx(item_i, item_i_)
    jidx = bidx(item_j, item_j_)
    gath = _gather(olds, e0, g1, g2, g3, uidx, iidx, jidx)

    nfb = _B // _FB
    dshape = (nfb, 16, _FB // 16)
    out = pl.pallas_call(
        _fin_body,
        grid=(nfb,),
        in_specs=[
            pl.BlockSpec((2, 3, _FB, _D), lambda b: (0, 0, b, 0)),
            pl.BlockSpec((1, 16, _FB // 16), lambda b: (b, 0, 0)),
            pl.BlockSpec((1, 16, _FB // 16), lambda b: (b, 0, 0)),
        ],
        out_specs=pl.BlockSpec((1, 1), lambda b: (0, 0)),
        out_shape=jax.ShapeDtypeStruct((1, 1), jnp.float32),
    )(gath.reshape(2, 3, _B, _D), degree_U.reshape(dshape),
      degree_I.reshape(dshape))
    return out.reshape(1)


# speedup vs baseline: 30.2223x; 1.0116x over previous
import functools

import jax
import jax.numpy as jnp
from jax import lax
from jax.experimental import pallas as pl
from jax.experimental.pallas import tpu as pltpu
from jax.experimental.pallas import tpu_sc as plsc

_N = 50000
_D = 32
_E = 1600000
_B = 16384
_T = 1.0

_SUB = 128
_NSUB = 3
_W = _SUB * _NSUB
_WINS = 268
_EPAD = 16 * _WINS * _W
_EROWS = _EPAD // _SUB

_RCH = 128
_NCHUNK = _N // _RCH
_NTAIL = _N - _NCHUNK * _RCH

_CH = 128
_CPT = _B // 16 // _CH

_mesh = plsc.VectorSubcoreMesh(core_axis_name="c", subcore_axis_name="s")


def _spmm_body(tabs, gidx, sidx, vals, out, acc, cbuf, sbuf, vbuf, gbuf,
               sem):
    c = lax.axis_index("c")
    s = lax.axis_index("s")

    def zrow(k, carry):
        gbuf[0, 0, k, pl.ds(0, 16)] = jnp.zeros((16,), jnp.float32)
        gbuf[0, 0, k, pl.ds(16, 16)] = jnp.zeros((16,), jnp.float32)
        return carry

    lax.fori_loop(0, _SUB, zrow, 0)

    def zcp(i, carry):
        cid = s + 16 * i

        @pl.when(cid < _NCHUNK)
        def _():
            r = pl.multiple_of(cid * _RCH, 8)
            pltpu.sync_copy(gbuf.at[0, 0], acc.at[pl.ds(r, _RCH)])

        @pl.when(cid == _NCHUNK)
        def _():
            pltpu.sync_copy(gbuf.at[0, 0, pl.ds(0, _NTAIL)],
                            acc.at[pl.ds(_NCHUNK * _RCH, _NTAIL)])

        return carry

    lax.fori_loop(0, 25, zcp, 0)
    plsc.subcore_barrier()

    sem_g = (sem.at[0], sem.at[1])
    sem_s = (sem.at[2], sem.at[3])
    sem_i = (sem.at[4], sem.at[5])

    def idx_fire(w, sl):
        r0 = (c * 16 + s) * _WINS * _NSUB + w * _NSUB
        rv = s * _WINS * _NSUB + w * _NSUB
        pltpu.async_copy(gidx.at[pl.ds(r0, _NSUB)], cbuf.at[sl], sem_i[sl])
        pltpu.async_copy(sidx.at[pl.ds(r0, _NSUB)], sbuf.at[sl], sem_i[sl])
        pltpu.async_copy(vals.at[pl.ds(rv, _NSUB)], vbuf.at[sl], sem_i[sl])

    def idx_drain(sl):
        pltpu.make_async_copy(gidx.at[pl.ds(0, _NSUB)], cbuf.at[sl],
                              sem_i[sl]).wait()
        pltpu.make_async_copy(sidx.at[pl.ds(0, _NSUB)], sbuf.at[sl],
                              sem_i[sl]).wait()
        pltpu.make_async_copy(vals.at[pl.ds(0, _NSUB)], vbuf.at[sl],
                              sem_i[sl]).wait()

    def gat_fire(sl):
        for j in range(_NSUB):
            pltpu.async_copy(tabs.at[cbuf.at[sl, j]], gbuf.at[sl, j],
                             sem_g[sl])

    def gat_drain(sl):
        for j in range(_NSUB):
            pltpu.make_async_copy(tabs.at[cbuf.at[sl, j]],
                                  gbuf.at[sl, j], sem_g[sl]).wait()

    def sca_fire(sl):
        for j in range(_NSUB):
            pltpu.async_copy(gbuf.at[sl, j], acc.at[sbuf.at[sl, j]],
                             sem_s[sl], add=True)

    def sca_drain(sl):
        for j in range(_NSUB):
            pltpu.make_async_copy(gbuf.at[sl, j], acc.at[sbuf.at[sl, j]],
                                  sem_s[sl]).wait()

    def scale(sl):
        for j in range(_NSUB):
            def sbody(k16, carry2, j=j):
                base = k16 * 16
                v16 = vbuf[sl, j, pl.ds(base, 16)]
                for g in range(2):
                    rows = [
                        (gbuf[sl, j, base + g * 8 + l, pl.ds(0, 16)],
                         gbuf[sl, j, base + g * 8 + l, pl.ds(16, 16)])
                        for l in range(8)
                    ]
                    for l in range(8):
                        vl = v16[g * 8 + l]
                        r = base + g * 8 + l
                        gbuf[sl, j, r, pl.ds(0, 16)] = rows[l][0] * vl
                        gbuf[sl, j, r, pl.ds(16, 16)] = rows[l][1] * vl
                return carry2

            lax.fori_loop(0, _SUB // 16, sbody, 0)

    def process(w, cur, nxt):
        @pl.when(w + 1 < _WINS)
        def _():
            idx_fire(w + 1, nxt)

        @pl.when(w >= 1)
        def _():
            sca_drain(nxt)

        @pl.when(w + 1 < _WINS)
        def _():
            idx_drain(nxt)
            gat_fire(nxt)

        gat_drain(cur)
        scale(cur)
        sca_fire(cur)

    idx_fire(0, 0)
    idx_drain(0)
    gat_fire(0)

    def wpair(i, carry):
        process(2 * i, 0, 1)
        process(2 * i + 1, 1, 0)
        return carry

    lax.fori_loop(0, _WINS // 2, wpair, 0)
    sca_drain(1)
    plsc.subcore_barrier()

    def obody(i, carry):
        cid = s + 16 * i

        @pl.when(cid < _NCHUNK)
        def _():
            r = pl.multiple_of(cid * _RCH, 8)
            ro = pl.multiple_of(c * _N + cid * _RCH, 8)
            pltpu.sync_copy(acc.at[pl.ds(r, _RCH)], gbuf.at[0, 0])
            pltpu.sync_copy(gbuf.at[0, 0], out.at[pl.ds(ro, _RCH)])

        @pl.when(cid == _NCHUNK)
        def _():
            ro = pl.multiple_of(c * _N + _NCHUNK * _RCH, 8)
            pltpu.sync_copy(acc.at[pl.ds(_NCHUNK * _RCH, _NTAIL)],
                            gbuf.at[0, 0, pl.ds(0, _NTAIL)])
            pltpu.sync_copy(gbuf.at[0, 0, pl.ds(0, _NTAIL)],
                            out.at[pl.ds(ro, _NTAIL)])

        return carry

    lax.fori_loop(0, 25, obody, 0)


_spmm = functools.partial(
    pl.kernel,
    out_type=jax.ShapeDtypeStruct((2 * _N, _D), jnp.float32),
    mesh=_mesh,
    scratch_types=[
        pltpu.VMEM_SHARED((_N, _D), jnp.float32),
        pltpu.VMEM((2, _NSUB, _SUB), jnp.int32),
        pltpu.VMEM((2, _NSUB, _SUB), jnp.int32),
        pltpu.VMEM((2, _NSUB, _SUB), jnp.float32),
        pltpu.VMEM((2, _NSUB, _SUB, _D), jnp.float32),
        pltpu.SemaphoreType.DMA((6,)),
    ],
    compiler_params=pltpu.CompilerParams(use_tc_tiling_on_sc=False,
                                         needs_layout_passes=False),
)(_spmm_body)


def _gather_body(olds, e0, g1, g2, g3, uidx, iidx, jidx, out,
                 ubi, ibi, jbi, ub, i0, i1, i2, i3, j0, j1, j2, j3,
                 sib, sjb, sem):
    c = lax.axis_index("c")
    s = lax.axis_index("s")
    r0 = pl.multiple_of(c * (_B // _CH) + s * _CPT, 8)
    pltpu.sync_copy(uidx.at[pl.ds(r0, _CPT)], ubi)
    pltpu.sync_copy(iidx.at[pl.ds(r0, _CPT)], ibi)
    pltpu.sync_copy(jidx.at[pl.ds(r0, _CPT)], jbi)

    def chunk(jj, carry):
        cps = [
            pltpu.async_copy(olds.at[ubi.at[jj]], ub, sem),
            pltpu.async_copy(e0.at[ibi.at[jj]], i0, sem),
            pltpu.async_copy(g1.at[ibi.at[jj]], i1, sem),
            pltpu.async_copy(g2.at[ibi.at[jj]], i2, sem),
            pltpu.async_copy(g3.at[ibi.at[jj]], i3, sem),
            pltpu.async_copy(e0.at[jbi.at[jj]], j0, sem),
            pltpu.async_copy(g1.at[jbi.at[jj]], j1, sem),
            pltpu.async_copy(g2.at[jbi.at[jj]], j2, sem),
            pltpu.async_copy(g3.at[jbi.at[jj]], j3, sem),
        ]
        for cp in cps:
            cp.wait()

        def comb(k, carry2):
            for sl in (pl.ds(0, 16), pl.ds(16, 16)):
                i0[k, sl] = (i0[k, sl] + 0.5 * i1[k, sl]
                             + (1.0 / 3.0) * i2[k, sl] + 0.25 * i3[k, sl])
                j0[k, sl] = (j0[k, sl] + 0.5 * j1[k, sl]
                             + (1.0 / 3.0) * j2[k, sl] + 0.25 * j3[k, sl])
            return carry2

        lax.fori_loop(0, _CH, comb, 0)

        def dots(k16, carry2):
            kvec = lax.iota(jnp.int32, 16) + k16 * 16
            zi = jnp.zeros((16,), jnp.float32)
            zj = jnp.zeros((16,), jnp.float32)
            for d in range(_D):
                dv = jnp.full((16,), 0, jnp.int32) + d
                uv = plsc.load_gather(ub, [kvec, dv])
                iv = plsc.load_gather(i0, [kvec, dv])
                jv = plsc.load_gather(j0, [kvec, dv])
                zi = zi + uv * iv
                zj = zj + uv * jv
            base = k16 * 16
            sib[pl.ds(base, 16)] = zi
            sjb[pl.ds(base, 16)] = zj
            return carry2

        lax.fori_loop(0, _CH // 16, dots, 0)
        b0 = (s * _CPT + jj) * _CH
        oi = pl.multiple_of((c * 2 + 0) * _B + b0, 8)
        oj = pl.multiple_of((c * 2 + 1) * _B + b0, 8)
        pltpu.sync_copy(sib, out.at[pl.ds(oi, _CH)])
        pltpu.sync_copy(sjb, out.at[pl.ds(oj, _CH)])
        return carry

    lax.fori_loop(0, _CPT, chunk, 0)


_gather = functools.partial(
    pl.kernel,
    out_type=jax.ShapeDtypeStruct((4 * _B,), jnp.float32),
    mesh=_mesh,
    scratch_types=(
        [pltpu.VMEM((_CPT, _CH), jnp.int32)] * 3
        + [pltpu.VMEM((_CH, _D), jnp.float32)] * 9
        + [pltpu.VMEM((_CH,), jnp.float32)] * 2
        + [pltpu.SemaphoreType.DMA]
    ),
    compiler_params=pltpu.CompilerParams(use_tc_tiling_on_sc=False,
                                         needs_layout_passes=False),
)(_gather_body)


_FB = 2048


def _fin_body(g_ref, du_ref, di_ref, out_ref):
    g = g_ref[...]
    deg = jnp.stack([du_ref[...], di_ref[...]])
    si = jnp.stack([g[0], g[2]]) / _T
    sj = jnp.stack([g[1], g[3]]) / _T
    num = jnp.exp(si)
    den = jnp.exp(sj) + num
    total = -jnp.sum(jnp.log(num / den) * deg) / _N
    out_ref[...] = jnp.reshape(total, (1, 1))


def kernel(user, item_i, item_j, degree_U, item_z_U, user_, item_i_, item_j_,
           degree_I, item_z_I, embed_user, embed_item, old_U_emb, old_I_emb,
           ui_rows, ui_cols, ui_vals):
    rows = ui_rows.astype(jnp.int32)
    cols = ui_cols.astype(jnp.int32)
    npad = _EPAD - _E
    pad_idx = jnp.arange(npad, dtype=jnp.int32) % _N
    pad_val = jnp.zeros((npad,), jnp.float32)
    cat = jnp.concatenate
    gidx = cat([cols + _N, pad_idx + _N, rows, pad_idx]).reshape(
        2 * _EROWS, _SUB)
    sidx = cat([rows, pad_idx, cols, pad_idx]).reshape(2 * _EROWS, _SUB)
    vals = cat([ui_vals.astype(jnp.float32), pad_val]).reshape(_EROWS, _SUB)

    e0 = cat([embed_user, embed_item])
    g1 = _spmm(e0, gidx, sidx, vals)
    g2 = _spmm(g1, gidx, sidx, vals)
    g3 = _spmm(g2, gidx, sidx, vals)

    olds = cat([old_U_emb, old_I_emb])
    i32 = jnp.int32

    def bidx(a, b):
        return cat([a.astype(i32), b.astype(i32) + _N]).reshape(
            2 * (_B // _CH), _CH)

    uidx = bidx(user, user_)
    iidx = bidx(item_i, item_i_)
    jidx = bidx(item_j, item_j_)
    gath = _gather(olds, e0, g1, g2, g3, uidx, iidx, jidx)

    out = pl.pallas_call(
        _fin_body,
        out_shape=jax.ShapeDtypeStruct((1, 1), jnp.float32),
    )(gath.reshape(4, 128, 128), degree_U.reshape(128, 128),
      degree_I.reshape(128, 128))
    return out.reshape(1)


# speedup vs baseline: 30.6393x; 1.0138x over previous
import functools

import jax
import jax.numpy as jnp
from jax import lax
from jax.experimental import pallas as pl
from jax.experimental.pallas import tpu as pltpu
from jax.experimental.pallas import tpu_sc as plsc

_N = 50000
_D = 32
_E = 1600000
_B = 16384
_T = 1.0

_SUB = 128
_NSUB = 3
_W = _SUB * _NSUB
_WINS = 268
_EPAD = 16 * _WINS * _W
_EROWS = _EPAD // _SUB

_RCH = 128
_NCHUNK = _N // _RCH
_NTAIL = _N - _NCHUNK * _RCH

_CH = 128
_CPT = _B // 16 // _CH

_mesh = plsc.VectorSubcoreMesh(core_axis_name="c", subcore_axis_name="s")


def _spmm_body(tabs, gidx, sidx, vals, out, acc, cbuf, sbuf, vbuf, gbuf,
               sem):
    c = lax.axis_index("c")
    s = lax.axis_index("s")

    def zrow(k, carry):
        gbuf[0, 0, k, pl.ds(0, 16)] = jnp.zeros((16,), jnp.float32)
        gbuf[0, 0, k, pl.ds(16, 16)] = jnp.zeros((16,), jnp.float32)
        return carry

    lax.fori_loop(0, _SUB, zrow, 0)

    def zfire(i, carry):
        cid = s + 16 * i

        @pl.when(cid < _NCHUNK)
        def _():
            r = pl.multiple_of(cid * _RCH, 8)
            pltpu.async_copy(gbuf.at[0, 0], acc.at[pl.ds(r, _RCH)], sem.at[0])

        @pl.when(cid == _NCHUNK)
        def _():
            pltpu.async_copy(gbuf.at[0, 0, pl.ds(0, _NTAIL)],
                             acc.at[pl.ds(_NCHUNK * _RCH, _NTAIL)], sem.at[0])

        return carry

    def zdrain(i, carry):
        cid = s + 16 * i

        @pl.when(cid < _NCHUNK)
        def _():
            r = pl.multiple_of(cid * _RCH, 8)
            pltpu.make_async_copy(gbuf.at[0, 0], acc.at[pl.ds(r, _RCH)],
                                  sem.at[0]).wait()

        @pl.when(cid == _NCHUNK)
        def _():
            pltpu.make_async_copy(gbuf.at[0, 0, pl.ds(0, _NTAIL)],
                                  acc.at[pl.ds(_NCHUNK * _RCH, _NTAIL)],
                                  sem.at[0]).wait()

        return carry

    lax.fori_loop(0, 25, zfire, 0)
    lax.fori_loop(0, 25, zdrain, 0)
    plsc.subcore_barrier()

    sem_g = (sem.at[0], sem.at[1])
    sem_s = (sem.at[2], sem.at[3])
    sem_i = (sem.at[4], sem.at[5])

    def idx_fire(w, sl):
        r0 = (c * 16 + s) * _WINS * _NSUB + w * _NSUB
        rv = s * _WINS * _NSUB + w * _NSUB
        pltpu.async_copy(gidx.at[pl.ds(r0, _NSUB)], cbuf.at[sl], sem_i[sl])
        pltpu.async_copy(sidx.at[pl.ds(r0, _NSUB)], sbuf.at[sl], sem_i[sl])
        pltpu.async_copy(vals.at[pl.ds(rv, _NSUB)], vbuf.at[sl], sem_i[sl])

    def idx_drain(sl):
        pltpu.make_async_copy(gidx.at[pl.ds(0, _NSUB)], cbuf.at[sl],
                              sem_i[sl]).wait()
        pltpu.make_async_copy(sidx.at[pl.ds(0, _NSUB)], sbuf.at[sl],
                              sem_i[sl]).wait()
        pltpu.make_async_copy(vals.at[pl.ds(0, _NSUB)], vbuf.at[sl],
                              sem_i[sl]).wait()

    def gat_fire(sl):
        for j in range(_NSUB):
            pltpu.async_copy(tabs.at[cbuf.at[sl, j]], gbuf.at[sl, j],
                             sem_g[sl])

    def gat_drain(sl):
        for j in range(_NSUB):
            pltpu.make_async_copy(tabs.at[cbuf.at[sl, j]],
                                  gbuf.at[sl, j], sem_g[sl]).wait()

    def sca_fire(sl):
        for j in range(_NSUB):
            pltpu.async_copy(gbuf.at[sl, j], acc.at[sbuf.at[sl, j]],
                             sem_s[sl], add=True)

    def sca_drain(sl):
        for j in range(_NSUB):
            pltpu.make_async_copy(gbuf.at[sl, j], acc.at[sbuf.at[sl, j]],
                                  sem_s[sl]).wait()

    def scale(sl):
        for j in range(_NSUB):
            def sbody(k16, carry2, j=j):
                base = k16 * 16
                v16 = vbuf[sl, j, pl.ds(base, 16)]
                for g in range(2):
                    rows = [
                        (gbuf[sl, j, base + g * 8 + l, pl.ds(0, 16)],
                         gbuf[sl, j, base + g * 8 + l, pl.ds(16, 16)])
                        for l in range(8)
                    ]
                    for l in range(8):
                        vl = v16[g * 8 + l]
                        r = base + g * 8 + l
                        gbuf[sl, j, r, pl.ds(0, 16)] = rows[l][0] * vl
                        gbuf[sl, j, r, pl.ds(16, 16)] = rows[l][1] * vl
                return carry2

            lax.fori_loop(0, _SUB // 16, sbody, 0)

    def process(w, cur, nxt):
        @pl.when(w + 1 < _WINS)
        def _():
            idx_fire(w + 1, nxt)

        @pl.when(w >= 1)
        def _():
            sca_drain(nxt)

        @pl.when(w + 1 < _WINS)
        def _():
            idx_drain(nxt)
            gat_fire(nxt)

        gat_drain(cur)
        scale(cur)
        sca_fire(cur)

    idx_fire(0, 0)
    idx_drain(0)
    gat_fire(0)

    def wpair(i, carry):
        process(2 * i, 0, 1)
        process(2 * i + 1, 1, 0)
        return carry

    lax.fori_loop(0, _WINS // 2, wpair, 0)
    sca_drain(1)
    plsc.subcore_barrier()

    def ostep(i, q):
        cid = s + 16 * i
        stage = gbuf.at[q, 0]

        @pl.when(cid <= _NCHUNK)
        def _():
            @pl.when(i >= 2)
            def _():
                pltpu.make_async_copy(stage, out.at[pl.ds(0, _RCH)],
                                      sem.at[q]).wait()

        @pl.when(cid < _NCHUNK)
        def _():
            r = pl.multiple_of(cid * _RCH, 8)
            ro = pl.multiple_of(c * _N + cid * _RCH, 8)
            pltpu.sync_copy(acc.at[pl.ds(r, _RCH)], stage)
            pltpu.async_copy(stage, out.at[pl.ds(ro, _RCH)], sem.at[q])

        @pl.when(cid == _NCHUNK)
        def _():
            ro = pl.multiple_of(c * _N + _NCHUNK * _RCH, 8)
            pltpu.sync_copy(acc.at[pl.ds(_NCHUNK * _RCH, _NTAIL)],
                            stage.at[pl.ds(0, _NTAIL)])
            pltpu.async_copy(stage.at[pl.ds(0, _NTAIL)],
                             out.at[pl.ds(ro, _NTAIL)], sem.at[q])

        return cid

    def opair(h, carry):
        ostep(2 * h, 0)
        i1 = 2 * h + 1

        @pl.when(i1 < 25)
        def _():
            ostep(i1, 1)

        return carry

    lax.fori_loop(0, 13, opair, 0)
    @pl.when(s == 6)
    def _():
        pltpu.make_async_copy(gbuf.at[0, 0, pl.ds(0, _NTAIL)],
                              out.at[pl.ds(0, _NTAIL)], sem.at[0]).wait()

    @pl.when(s != 6)
    def _():
        pltpu.make_async_copy(gbuf.at[0, 0], out.at[pl.ds(0, _RCH)],
                              sem.at[0]).wait()

    pltpu.make_async_copy(gbuf.at[1, 0], out.at[pl.ds(0, _RCH)],
                          sem.at[1]).wait()


_spmm = functools.partial(
    pl.kernel,
    out_type=jax.ShapeDtypeStruct((2 * _N, _D), jnp.float32),
    mesh=_mesh,
    scratch_types=[
        pltpu.VMEM_SHARED((_N, _D), jnp.float32),
        pltpu.VMEM((2, _NSUB, _SUB), jnp.int32),
        pltpu.VMEM((2, _NSUB, _SUB), jnp.int32),
        pltpu.VMEM((2, _NSUB, _SUB), jnp.float32),
        pltpu.VMEM((2, _NSUB, _SUB, _D), jnp.float32),
        pltpu.SemaphoreType.DMA((6,)),
    ],
    compiler_params=pltpu.CompilerParams(use_tc_tiling_on_sc=False,
                                         needs_layout_passes=False),
)(_spmm_body)


def _gather_body(olds, e0, g1, g2, g3, uidx, iidx, jidx, out,
                 ubi, ibi, jbi, ub, i0, i1, i2, i3, j0, j1, j2, j3,
                 sib, sjb, sem):
    c = lax.axis_index("c")
    s = lax.axis_index("s")
    r0 = pl.multiple_of(c * (_B // _CH) + s * _CPT, 8)
    pltpu.sync_copy(uidx.at[pl.ds(r0, _CPT)], ubi)
    pltpu.sync_copy(iidx.at[pl.ds(r0, _CPT)], ibi)
    pltpu.sync_copy(jidx.at[pl.ds(r0, _CPT)], jbi)

    def chunk(jj, carry):
        cps = [
            pltpu.async_copy(olds.at[ubi.at[jj]], ub, sem),
            pltpu.async_copy(e0.at[ibi.at[jj]], i0, sem),
            pltpu.async_copy(g1.at[ibi.at[jj]], i1, sem),
            pltpu.async_copy(g2.at[ibi.at[jj]], i2, sem),
            pltpu.async_copy(g3.at[ibi.at[jj]], i3, sem),
            pltpu.async_copy(e0.at[jbi.at[jj]], j0, sem),
            pltpu.async_copy(g1.at[jbi.at[jj]], j1, sem),
            pltpu.async_copy(g2.at[jbi.at[jj]], j2, sem),
            pltpu.async_copy(g3.at[jbi.at[jj]], j3, sem),
        ]
        for cp in cps:
            cp.wait()

        def comb(k, carry2):
            for sl in (pl.ds(0, 16), pl.ds(16, 16)):
                i0[k, sl] = (i0[k, sl] + 0.5 * i1[k, sl]
                             + (1.0 / 3.0) * i2[k, sl] + 0.25 * i3[k, sl])
                j0[k, sl] = (j0[k, sl] + 0.5 * j1[k, sl]
                             + (1.0 / 3.0) * j2[k, sl] + 0.25 * j3[k, sl])
            return carry2

        lax.fori_loop(0, _CH, comb, 0)

        def dots(k16, carry2):
            kvec = lax.iota(jnp.int32, 16) + k16 * 16
            zi = jnp.zeros((16,), jnp.float32)
            zj = jnp.zeros((16,), jnp.float32)
            for d in range(_D):
                dv = jnp.full((16,), 0, jnp.int32) + d
                uv = plsc.load_gather(ub, [kvec, dv])
                iv = plsc.load_gather(i0, [kvec, dv])
                jv = plsc.load_gather(j0, [kvec, dv])
                zi = zi + uv * iv
                zj = zj + uv * jv
            base = k16 * 16
            sib[pl.ds(base, 16)] = zi
            sjb[pl.ds(base, 16)] = zj
            return carry2

        lax.fori_loop(0, _CH // 16, dots, 0)
        b0 = (s * _CPT + jj) * _CH
        oi = pl.multiple_of((c * 2 + 0) * _B + b0, 8)
        oj = pl.multiple_of((c * 2 + 1) * _B + b0, 8)
        pltpu.sync_copy(sib, out.at[pl.ds(oi, _CH)])
        pltpu.sync_copy(sjb, out.at[pl.ds(oj, _CH)])
        return carry

    lax.fori_loop(0, _CPT, chunk, 0)


_gather = functools.partial(
    pl.kernel,
    out_type=jax.ShapeDtypeStruct((4 * _B,), jnp.float32),
    mesh=_mesh,
    scratch_types=(
        [pltpu.VMEM((_CPT, _CH), jnp.int32)] * 3
        + [pltpu.VMEM((_CH, _D), jnp.float32)] * 9
        + [pltpu.VMEM((_CH,), jnp.float32)] * 2
        + [pltpu.SemaphoreType.DMA]
    ),
    compiler_params=pltpu.CompilerParams(use_tc_tiling_on_sc=False,
                                         needs_layout_passes=False),
)(_gather_body)


_FB = 2048


def _fin_body(g_ref, du_ref, di_ref, out_ref):
    g = g_ref[...]
    deg = jnp.stack([du_ref[...], di_ref[...]])
    si = jnp.stack([g[0], g[2]]) / _T
    sj = jnp.stack([g[1], g[3]]) / _T
    num = jnp.exp(si)
    den = jnp.exp(sj) + num
    total = -jnp.sum(jnp.log(num / den) * deg) / _N
    out_ref[...] = jnp.reshape(total, (1, 1))


def kernel(user, item_i, item_j, degree_U, item_z_U, user_, item_i_, item_j_,
           degree_I, item_z_I, embed_user, embed_item, old_U_emb, old_I_emb,
           ui_rows, ui_cols, ui_vals):
    rows = ui_rows.astype(jnp.int32)
    cols = ui_cols.astype(jnp.int32)
    npad = _EPAD - _E
    pad_idx = jnp.arange(npad, dtype=jnp.int32) % _N
    pad_val = jnp.zeros((npad,), jnp.float32)
    cat = jnp.concatenate
    gidx = cat([cols + _N, pad_idx + _N, rows, pad_idx]).reshape(
        2 * _EROWS, _SUB)
    sidx = cat([rows, pad_idx, cols, pad_idx]).reshape(2 * _EROWS, _SUB)
    vals = cat([ui_vals.astype(jnp.float32), pad_val]).reshape(_EROWS, _SUB)

    e0 = cat([embed_user, embed_item])
    g1 = _spmm(e0, gidx, sidx, vals)
    g2 = _spmm(g1, gidx, sidx, vals)
    g3 = _spmm(g2, gidx, sidx, vals)

    olds = cat([old_U_emb, old_I_emb])
    i32 = jnp.int32

    def bidx(a, b):
        return cat([a.astype(i32), b.astype(i32) + _N]).reshape(
            2 * (_B // _CH), _CH)

    uidx = bidx(user, user_)
    iidx = bidx(item_i, item_i_)
    jidx = bidx(item_j, item_j_)
    gath = _gather(olds, e0, g1, g2, g3, uidx, iidx, jidx)

    out = pl.pallas_call(
        _fin_body,
        out_shape=jax.ShapeDtypeStruct((1, 1), jnp.float32),
    )(gath.reshape(4, 128, 128), degree_U.reshape(128, 128),
      degree_I.reshape(128, 128))
    return out.reshape(1)
